# Initial kernel scaffold; baseline (speedup 1.0000x reference)
import jax
import jax.numpy as jnp
from jax.experimental import pallas as pl


def kernel(x_user, x_product, ei_reviews, ei_rev_reviews, ei_similar, w1_rp_l, b1_rp, w1_rp_r, w1_pu_l, b1_pu, w1_pu_r, w1_pp_l, b1_pp, w1_pp_r, w2_rp_l, b2_rp, w2_rp_r, w2_pu_l, b2_pu, w2_pu_r, w2_pp_l, b2_pp, w2_pp_r):
    raise NotImplementedError("write your pallas kernel here")


# speedup vs baseline: 2.4184x; 2.4184x over previous
import jax
import jax.numpy as jnp
from jax import lax
from jax.experimental import pallas as pl
from jax.experimental.pallas import tpu as pltpu
from jax.experimental.pallas import tpu_sc as plsc

_NU = 50000
_NP = 50000
_BN = 2000
_CH = 4
_NSUB = 16
_NACC = 50048
_TRASH = 50000
_WPS = 3128
_ZPS = _NACC // _NSUB


def _zero_acc(acc, zbuf, sid):
    def zacc(t, carry):
        pltpu.sync_copy(zbuf, acc.at[pl.ds(sid * _ZPS + t * 128, 128)])
        return carry
    lax.fori_loop(0, _ZPS // 128, zacc, 0)
    rem = _ZPS % 128
    pltpu.sync_copy(zbuf.at[pl.ds(0, rem)],
                    acc.at[pl.ds(sid * _ZPS + _ZPS - rem, rem)])


def _writeout(acc, out_ref, sid, out_base, n_dst):
    last = n_dst - _WPS

    @pl.when(sid < _NSUB - 1)
    def _():
        b = pl.multiple_of(sid * _WPS, 8)
        pltpu.sync_copy(acc.at[pl.ds(b, _WPS)],
                        out_ref.at[pl.ds(pl.multiple_of(out_base + b, 8),
                                         _WPS)])

    @pl.when(sid == _NSUB - 1)
    def _():
        pltpu.sync_copy(acc.at[pl.ds(last, _WPS)],
                        out_ref.at[pl.ds(pl.multiple_of(out_base + last, 8),
                                         _WPS)])


def _prep(src, dst):
    e = src.shape[0]
    nb = ((e + 32767) // 32768) * 256
    ep = nb * 128
    src = jnp.concatenate([src.astype(jnp.int32),
                           jnp.zeros((ep - e,), jnp.int32)])
    dst = jnp.concatenate([dst.astype(jnp.int32),
                           jnp.full((ep - e,), _TRASH, jnp.int32)])
    return src.reshape(nb, 128), dst.reshape(nb, 128)


def _adj(src2d, j_count, n_src):
    off = (jnp.arange(j_count, dtype=jnp.int32) * n_src)[:, None, None]
    return (src2d[None] + off).reshape(j_count * src2d.shape[0], 128)


_MESH = dict(core_axis_name="c", subcore_axis_name="s")


def _sc_agg(table, srcadj, dst2d, j_count, n_dst):
    nb = dst2d.shape[0]
    gps = nb // _NSUB
    nbatch = gps // _CH
    rounds = j_count // 2

    def body(table_ref, src_ref, dst_ref, out_ref, acc, srcbuf, dstbuf,
             r0, r1, r2, r3, gsem):
        rows = (r0, r1, r2, r3)
        cid = lax.axis_index("c")
        sid = lax.axis_index("s")

        for r in range(rounds):
            j = cid + 2 * r

            def zrow(i, carry):
                r0[i, pl.ds(0, 16)] = jnp.zeros((16,), jnp.float32)
                r0[i, pl.ds(16, 16)] = jnp.zeros((16,), jnp.float32)
                return carry
            lax.fori_loop(0, 128, zrow, 0)
            _zero_acc(acc, r0, sid)
            plsc.subcore_barrier()

            def batch(q, carry):
                gb = sid * gps + q * _CH
                pltpu.sync_copy(src_ref.at[pl.ds(j * nb + gb, _CH)], srcbuf)
                pltpu.sync_copy(dst_ref.at[pl.ds(gb, _CH)], dstbuf)
                descs = [pltpu.async_copy(table_ref.at[srcbuf.at[b]], rows[b],
                                          gsem)
                         for b in range(_CH)]
                for d in descs:
                    d.wait()
                for b in range(_CH):
                    pltpu.sync_copy(rows[b], acc.at[dstbuf.at[b]], add=True)
                return carry
            lax.fori_loop(0, nbatch, batch, 0)
            plsc.subcore_barrier()

            _writeout(acc, out_ref, sid, j * n_dst, n_dst)
            plsc.subcore_barrier()

    f = pl.kernel(
        body,
        out_type=jax.ShapeDtypeStruct((j_count * n_dst, 32), jnp.float32),
        mesh=plsc.VectorSubcoreMesh(**_MESH),
        compiler_params=pltpu.CompilerParams(use_tc_tiling_on_sc=False),
        scratch_types=(
            [pltpu.VMEM_SHARED((_NACC, 32), jnp.float32),
             pltpu.VMEM((_CH, 128), jnp.int32),
             pltpu.VMEM((_CH, 128), jnp.int32)]
            + [pltpu.VMEM((128, 32), jnp.float32) for _ in range(_CH)]
            + [pltpu.SemaphoreType.DMA]
        ),
    )
    return f(table, srcadj, dst2d).reshape(j_count, n_dst, 32)


def _sc_counts(dst_r, dst_v, dst_s):
    n = _NP

    def body(dr_ref, dv_ref, ds_ref, out_ref, cacc, dstbuf, ones):
        cid = lax.axis_index("c")
        sid = lax.axis_index("s")
        wid = cid * _NSUB + sid

        for t, dref in ((0, dr_ref), (1, dv_ref), (2, ds_ref)):
            nb = dref.shape[0]
            gps = nb // (2 * _NSUB)
            nbatch = gps // _CH

            def zrow(i, carry):
                ones[i, pl.ds(0, 16)] = jnp.zeros((16,), jnp.float32)
                return carry
            lax.fori_loop(0, 128, zrow, 0)
            _zero_acc(cacc, ones, sid)

            def frow(i, carry):
                ones[i, pl.ds(0, 16)] = jnp.ones((16,), jnp.float32)
                return carry
            lax.fori_loop(0, 128, frow, 0)
            plsc.subcore_barrier()

            def batch(q, carry):
                gb = wid * gps + q * _CH
                pltpu.sync_copy(dref.at[pl.ds(gb, _CH)], dstbuf)
                for b in range(_CH):
                    pltpu.sync_copy(ones, cacc.at[dstbuf.at[b]], add=True)
                return carry
            lax.fori_loop(0, nbatch, batch, 0)
            plsc.subcore_barrier()

            _writeout(cacc, out_ref, sid, cid * 3 * n + t * n, n)
            plsc.subcore_barrier()

    f = pl.kernel(
        body,
        out_type=jax.ShapeDtypeStruct((2 * 3 * n, 16), jnp.float32),
        mesh=plsc.VectorSubcoreMesh(**_MESH),
        compiler_params=pltpu.CompilerParams(use_tc_tiling_on_sc=False),
        scratch_types=[
            pltpu.VMEM_SHARED((_NACC, 16), jnp.float32),
            pltpu.VMEM((_CH, 128), jnp.int32),
            pltpu.VMEM((128, 16), jnp.float32),
        ],
    )
    return f(dst_r, dst_v, dst_s).reshape(2, 3, n, 16)


def _tc_mm_blocked(x, w):
    n, k = x.shape
    wo = w.shape[1]
    jc = wo // 32

    def body(x_ref, w_ref, o_ref):
        y = jnp.dot(x_ref[...], w_ref[...], preferred_element_type=jnp.float32)
        for t in range(jc):
            o_ref[t] = y[:, t * 32:(t + 1) * 32]

    return pl.pallas_call(
        body,
        grid=(n // _BN,),
        in_specs=[pl.BlockSpec((_BN, k), lambda i: (i, 0)),
                  pl.BlockSpec((k, wo), lambda i: (0, 0))],
        out_specs=pl.BlockSpec((jc, _BN, 32), lambda i: (0, i, 0)),
        out_shape=jax.ShapeDtypeStruct((jc, n, 32), jnp.float32),
    )(x, w)


def _tc_mm_from_blocked(hb, w):
    n = hb.shape[1]
    wo = w.shape[1]
    jc = wo // 32

    def body(h_ref, w_ref, o_ref):
        h = jnp.concatenate([h_ref[t] for t in range(4)], axis=1)
        y = jnp.dot(h, w_ref[...], preferred_element_type=jnp.float32)
        for t in range(jc):
            o_ref[t] = y[:, t * 32:(t + 1) * 32]

    return pl.pallas_call(
        body,
        grid=(n // _BN,),
        in_specs=[pl.BlockSpec((4, _BN, 32), lambda i: (0, i, 0)),
                  pl.BlockSpec((128, wo), lambda i: (0, 0))],
        out_specs=pl.BlockSpec((jc, _BN, 32), lambda i: (0, i, 0)),
        out_shape=jax.ShapeDtypeStruct((jc, n, 32), jnp.float32),
    )(hb, w)


def _inv_cnt(c0, c1):
    return 1.0 / jnp.maximum(c0[:, 0:1] + c1[:, 0:1], 1.0)


def _tc_combine1_dual(agg_a, ca0, ca1, agg_b, cb0, cb1, x, wa, wb, ba, bb):
    n = x.shape[0]

    def body(aa, a0, a1, ab, b0, b1, x_ref, wa_ref, wb_ref, ba_ref, bb_ref,
             o_ref):
        sa = jnp.concatenate([aa[t] for t in range(4)], axis=1)
        sb = jnp.concatenate([ab[t] for t in range(4)], axis=1)
        y = jnp.dot(x_ref[...], wa_ref[...] + wb_ref[...],
                    preferred_element_type=jnp.float32)
        res = (sa * _inv_cnt(a0, a1) + sb * _inv_cnt(b0, b1) + y
               + ba_ref[...] + bb_ref[...])
        res = jnp.maximum(res, 0.0)
        for t in range(4):
            o_ref[t] = res[:, t * 32:(t + 1) * 32]

    cspec = pl.BlockSpec((_BN, 16), lambda i: (i, 0))
    aspec = pl.BlockSpec((4, _BN, 32), lambda i: (0, i, 0))
    wspec = pl.BlockSpec((128, 128), lambda i: (0, 0))
    bspec = pl.BlockSpec((1, 128), lambda i: (0, 0))
    return pl.pallas_call(
        body,
        grid=(n // _BN,),
        in_specs=[aspec, cspec, cspec, aspec, cspec, cspec,
                  pl.BlockSpec((_BN, 128), lambda i: (i, 0)),
                  wspec, wspec, bspec, bspec],
        out_specs=pl.BlockSpec((4, _BN, 32), lambda i: (0, i, 0)),
        out_shape=jax.ShapeDtypeStruct((4, n, 32), jnp.float32),
    )(agg_a, ca0, ca1, agg_b, cb0, cb1, x, wa, wb, ba, bb)


def _tc_combine1_single(agg_a, ca0, ca1, x, wa, ba):
    n = x.shape[0]

    def body(aa, a0, a1, x_ref, wa_ref, ba_ref, o_ref):
        sa = jnp.concatenate([aa[t] for t in range(4)], axis=1)
        y = jnp.dot(x_ref[...], wa_ref[...],
                    preferred_element_type=jnp.float32)
        res = jnp.maximum(sa * _inv_cnt(a0, a1) + y + ba_ref[...], 0.0)
        for t in range(4):
            o_ref[t] = res[:, t * 32:(t + 1) * 32]

    cspec = pl.BlockSpec((_BN, 16), lambda i: (i, 0))
    return pl.pallas_call(
        body,
        grid=(n // _BN,),
        in_specs=[pl.BlockSpec((4, _BN, 32), lambda i: (0, i, 0)),
                  cspec, cspec,
                  pl.BlockSpec((_BN, 128), lambda i: (i, 0)),
                  pl.BlockSpec((128, 128), lambda i: (0, 0)),
                  pl.BlockSpec((1, 128), lambda i: (0, 0))],
        out_specs=pl.BlockSpec((4, _BN, 32), lambda i: (0, i, 0)),
        out_shape=jax.ShapeDtypeStruct((4, n, 32), jnp.float32),
    )(agg_a, ca0, ca1, x, wa, ba)


def _tc_combine2_dual(agg_a, ca0, ca1, agg_b, cb0, cb1, hb, wa, wb, ba, bb):
    n = hb.shape[1]

    def body(aa, a0, a1, ab, b0, b1, h_ref, wa_ref, wb_ref, ba_ref, bb_ref,
             o_ref):
        sa = jnp.concatenate([aa[0], aa[1]], axis=1)
        sb = jnp.concatenate([ab[0], ab[1]], axis=1)
        h = jnp.concatenate([h_ref[t] for t in range(4)], axis=1)
        y = jnp.dot(h, wa_ref[...] + wb_ref[...],
                    preferred_element_type=jnp.float32)
        o_ref[...] = (sa * _inv_cnt(a0, a1) + sb * _inv_cnt(b0, b1) + y
                      + ba_ref[...] + bb_ref[...])

    cspec = pl.BlockSpec((_BN, 16), lambda i: (i, 0))
    aspec = pl.BlockSpec((2, _BN, 32), lambda i: (0, i, 0))
    wspec = pl.BlockSpec((128, 64), lambda i: (0, 0))
    bspec = pl.BlockSpec((1, 64), lambda i: (0, 0))
    return pl.pallas_call(
        body,
        grid=(n // _BN,),
        in_specs=[aspec, cspec, cspec, aspec, cspec, cspec,
                  pl.BlockSpec((4, _BN, 32), lambda i: (0, i, 0)),
                  wspec, wspec, bspec, bspec],
        out_specs=pl.BlockSpec((_BN, 64), lambda i: (i, 0)),
        out_shape=jax.ShapeDtypeStruct((n, 64), jnp.float32),
    )(agg_a, ca0, ca1, agg_b, cb0, cb1, hb, wa, wb, ba, bb)


def _tc_combine2_single(agg_a, ca0, ca1, hb, wa, ba):
    n = hb.shape[1]

    def body(aa, a0, a1, h_ref, wa_ref, ba_ref, o_ref):
        sa = jnp.concatenate([aa[0], aa[1]], axis=1)
        h = jnp.concatenate([h_ref[t] for t in range(4)], axis=1)
        y = jnp.dot(h, wa_ref[...], preferred_element_type=jnp.float32)
        o_ref[...] = sa * _inv_cnt(a0, a1) + y + ba_ref[...]

    cspec = pl.BlockSpec((_BN, 16), lambda i: (i, 0))
    return pl.pallas_call(
        body,
        grid=(n // _BN,),
        in_specs=[pl.BlockSpec((2, _BN, 32), lambda i: (0, i, 0)),
                  cspec, cspec,
                  pl.BlockSpec((4, _BN, 32), lambda i: (0, i, 0)),
                  pl.BlockSpec((128, 64), lambda i: (0, 0)),
                  pl.BlockSpec((1, 64), lambda i: (0, 0))],
        out_specs=pl.BlockSpec((_BN, 64), lambda i: (i, 0)),
        out_shape=jax.ShapeDtypeStruct((n, 64), jnp.float32),
    )(agg_a, ca0, ca1, hb, wa, ba)


def kernel(x_user, x_product, ei_reviews, ei_rev_reviews, ei_similar,
           w1_rp_l, b1_rp, w1_rp_r, w1_pu_l, b1_pu, w1_pu_r,
           w1_pp_l, b1_pp, w1_pp_r,
           w2_rp_l, b2_rp, w2_rp_r, w2_pu_l, b2_pu, w2_pu_r,
           w2_pp_l, b2_pp, w2_pp_r):
    s_r, d_r = _prep(ei_reviews[0], ei_reviews[1])
    s_v, d_v = _prep(ei_rev_reviews[0], ei_rev_reviews[1])
    s_s, d_s = _prep(ei_similar[0], ei_similar[1])

    cparts = _sc_counts(d_r, d_v, d_s)
    cr0, cr1 = cparts[0, 0], cparts[1, 0]
    cv0, cv1 = cparts[0, 1], cparts[1, 1]
    cs0, cs1 = cparts[0, 2], cparts[1, 2]

    yu1 = _tc_mm_blocked(x_user, w1_rp_l)
    ypp1 = _tc_mm_blocked(x_product, w1_pp_l)
    ypu1 = _tc_mm_blocked(x_product, w1_pu_l)
    agg_r1 = _sc_agg(yu1.reshape(-1, 32), _adj(s_r, 4, _NU), d_r, 4, _NP)
    agg_s1 = _sc_agg(ypp1.reshape(-1, 32), _adj(s_s, 4, _NP), d_s, 4, _NP)
    agg_v1 = _sc_agg(ypu1.reshape(-1, 32), _adj(s_v, 4, _NP), d_v, 4, _NU)
    h_p = _tc_combine1_dual(agg_r1, cr0, cr1, agg_s1, cs0, cs1, x_product,
                            w1_rp_r, w1_pp_r,
                            b1_rp.reshape(1, -1), b1_pp.reshape(1, -1))
    h_u = _tc_combine1_single(agg_v1, cv0, cv1, x_user, w1_pu_r,
                              b1_pu.reshape(1, -1))

    y2r = _tc_mm_from_blocked(h_u, w2_rp_l)
    y2s = _tc_mm_from_blocked(h_p, w2_pp_l)
    y2v = _tc_mm_from_blocked(h_p, w2_pu_l)
    agg_r2 = _sc_agg(y2r.reshape(-1, 32), _adj(s_r, 2, _NU), d_r, 2, _NP)
    agg_s2 = _sc_agg(y2s.reshape(-1, 32), _adj(s_s, 2, _NP), d_s, 2, _NP)
    agg_v2 = _sc_agg(y2v.reshape(-1, 32), _adj(s_v, 2, _NP), d_v, 2, _NU)
    out_p = _tc_combine2_dual(agg_r2, cr0, cr1, agg_s2, cs0, cs1, h_p,
                              w2_rp_r, w2_pp_r,
                              b2_rp.reshape(1, -1), b2_pp.reshape(1, -1))
    out_u = _tc_combine2_single(agg_v2, cv0, cv1, h_u, w2_pu_r,
                                b2_pu.reshape(1, -1))
    return (out_u, out_p)


# speedup vs baseline: 2.5258x; 1.0444x over previous
import jax
import jax.numpy as jnp
from jax import lax
from jax.experimental import pallas as pl
from jax.experimental.pallas import tpu as pltpu
from jax.experimental.pallas import tpu_sc as plsc

_NU = 50000
_NP = 50000
_BN = 2000
_NBIG = 16
_NSUB = 16
_NACC = 50048
_TRASH = 50000
_WPS = 3128
_ZPS = _NACC // _NSUB


def _zero_acc(acc, zbuf, sid):
    def zacc(t, carry):
        pltpu.sync_copy(zbuf, acc.at[pl.ds(sid * _ZPS + t * 128, 128)])
        return carry
    lax.fori_loop(0, _ZPS // 128, zacc, 0)
    rem = _ZPS % 128
    pltpu.sync_copy(zbuf.at[pl.ds(0, rem)],
                    acc.at[pl.ds(sid * _ZPS + _ZPS - rem, rem)])


def _writeout(acc, out_ref, sid, out_base, n_dst):
    last = n_dst - _WPS

    @pl.when(sid < _NSUB - 1)
    def _():
        b = pl.multiple_of(sid * _WPS, 8)
        pltpu.sync_copy(acc.at[pl.ds(b, _WPS)],
                        out_ref.at[pl.ds(pl.multiple_of(out_base + b, 8),
                                         _WPS)])

    @pl.when(sid == _NSUB - 1)
    def _():
        pltpu.sync_copy(acc.at[pl.ds(last, _WPS)],
                        out_ref.at[pl.ds(pl.multiple_of(out_base + last, 8),
                                         _WPS)])


def _prep(src, dst):
    e = src.shape[0]
    nb = ((e + 32767) // 32768) * 256
    ep = nb * 128
    src = jnp.concatenate([src.astype(jnp.int32),
                           jnp.zeros((ep - e,), jnp.int32)])
    dst = jnp.concatenate([dst.astype(jnp.int32),
                           jnp.full((ep - e,), _TRASH, jnp.int32)])
    return src.reshape(nb, 128), dst.reshape(nb, 128)


def _adj(src2d, j_count, n_src):
    off = (jnp.arange(j_count, dtype=jnp.int32) * n_src)[:, None, None]
    return (src2d[None] + off).reshape(j_count * src2d.shape[0], 128)


_MESH = dict(core_axis_name="c", subcore_axis_name="s")


def _sc_agg(table, srcadj, dst2d, j_count, n_dst):
    nb = dst2d.shape[0]
    gps = nb // _NSUB
    nchunk = gps // _NBIG
    rounds = j_count // 2

    def body(table_ref, src_ref, dst_ref, out_ref, acc, srcbig, dstbig,
             r0, r1, r2, r3, gsem, ssem_a, ssem_b):
        cid = lax.axis_index("c")
        sid = lax.axis_index("s")

        for r in range(rounds):
            j = cid + 2 * r

            def zrow(i, carry):
                r0[i, pl.ds(0, 16)] = jnp.zeros((16,), jnp.float32)
                r0[i, pl.ds(16, 16)] = jnp.zeros((16,), jnp.float32)
                return carry
            lax.fori_loop(0, 128, zrow, 0)
            _zero_acc(acc, r0, sid)
            plsc.subcore_barrier()

            def chunk(cq, carry):
                base = sid * gps + cq * _NBIG
                pltpu.sync_copy(src_ref.at[pl.ds(j * nb + base, _NBIG)],
                                srcbig)
                pltpu.sync_copy(dst_ref.at[pl.ds(base, _NBIG)], dstbig)

                def pair(p, carry2):
                    for ra, rb, sem, k in ((r0, r1, ssem_a, 4 * p),
                                           (r2, r3, ssem_b, 4 * p + 2)):
                        @pl.when(p > 0)
                        def _():
                            pltpu.make_async_copy(
                                ra, acc.at[dstbig.at[k]], sem).wait()
                            pltpu.make_async_copy(
                                rb, acc.at[dstbig.at[k]], sem).wait()
                        da = pltpu.async_copy(table_ref.at[srcbig.at[k]],
                                              ra, gsem)
                        db = pltpu.async_copy(table_ref.at[srcbig.at[k + 1]],
                                              rb, gsem)
                        da.wait()
                        db.wait()
                        pltpu.async_copy(ra, acc.at[dstbig.at[k]], sem,
                                         add=True)
                        pltpu.async_copy(rb, acc.at[dstbig.at[k + 1]], sem,
                                         add=True)
                    return carry2
                lax.fori_loop(0, _NBIG // 4, pair, 0)
                pltpu.make_async_copy(r0, acc.at[dstbig.at[0]], ssem_a).wait()
                pltpu.make_async_copy(r1, acc.at[dstbig.at[0]], ssem_a).wait()
                pltpu.make_async_copy(r2, acc.at[dstbig.at[0]], ssem_b).wait()
                pltpu.make_async_copy(r3, acc.at[dstbig.at[0]], ssem_b).wait()
                return carry
            lax.fori_loop(0, nchunk, chunk, 0)
            plsc.subcore_barrier()

            _writeout(acc, out_ref, sid, j * n_dst, n_dst)
            plsc.subcore_barrier()

    f = pl.kernel(
        body,
        out_type=jax.ShapeDtypeStruct((j_count * n_dst, 32), jnp.float32),
        mesh=plsc.VectorSubcoreMesh(**_MESH),
        compiler_params=pltpu.CompilerParams(use_tc_tiling_on_sc=False),
        scratch_types=(
            [pltpu.VMEM_SHARED((_NACC, 32), jnp.float32),
             pltpu.VMEM((_NBIG, 128), jnp.int32),
             pltpu.VMEM((_NBIG, 128), jnp.int32)]
            + [pltpu.VMEM((128, 32), jnp.float32) for _ in range(4)]
            + [pltpu.SemaphoreType.DMA] * 3
        ),
    )
    return f(table, srcadj, dst2d).reshape(j_count, n_dst, 32)


def _sc_counts(dst_r, dst_v, dst_s):
    n = _NP

    def body(dr_ref, dv_ref, ds_ref, out_ref, cacc, dstbig, ones, ssem):
        cid = lax.axis_index("c")
        sid = lax.axis_index("s")
        wid = cid * _NSUB + sid

        for t, dref in ((0, dr_ref), (1, dv_ref), (2, ds_ref)):
            nb = dref.shape[0]
            gps = nb // (2 * _NSUB)
            nchunk = gps // _NBIG

            def zrow(i, carry):
                ones[i, pl.ds(0, 16)] = jnp.zeros((16,), jnp.float32)
                return carry
            lax.fori_loop(0, 128, zrow, 0)
            _zero_acc(cacc, ones, sid)

            def frow(i, carry):
                ones[i, pl.ds(0, 16)] = jnp.ones((16,), jnp.float32)
                return carry
            lax.fori_loop(0, 128, frow, 0)
            plsc.subcore_barrier()

            def chunk(cq, carry):
                gb = wid * gps + cq * _NBIG
                pltpu.sync_copy(dref.at[pl.ds(gb, _NBIG)], dstbig)

                def quad(p, carry2):
                    for i in range(4):
                        pltpu.async_copy(ones, cacc.at[dstbig.at[4 * p + i]],
                                         ssem, add=True)
                    return carry2
                lax.fori_loop(0, _NBIG // 4, quad, 0)
                for _ in range(_NBIG):
                    pltpu.make_async_copy(ones, cacc.at[dstbig.at[0]],
                                          ssem).wait()
                return carry
            lax.fori_loop(0, nchunk, chunk, 0)
            plsc.subcore_barrier()

            _writeout(cacc, out_ref, sid, cid * 3 * n + t * n, n)
            plsc.subcore_barrier()

    f = pl.kernel(
        body,
        out_type=jax.ShapeDtypeStruct((2 * 3 * n, 16), jnp.float32),
        mesh=plsc.VectorSubcoreMesh(**_MESH),
        compiler_params=pltpu.CompilerParams(use_tc_tiling_on_sc=False),
        scratch_types=[
            pltpu.VMEM_SHARED((_NACC, 16), jnp.float32),
            pltpu.VMEM((_NBIG, 128), jnp.int32),
            pltpu.VMEM((128, 16), jnp.float32),
            pltpu.SemaphoreType.DMA,
        ],
    )
    return f(dst_r, dst_v, dst_s).reshape(2, 3, n, 16)


def _tc_mm_blocked(x, w):
    n, k = x.shape
    wo = w.shape[1]
    jc = wo // 32

    def body(x_ref, w_ref, o_ref):
        y = jnp.dot(x_ref[...], w_ref[...], preferred_element_type=jnp.float32)
        for t in range(jc):
            o_ref[t] = y[:, t * 32:(t + 1) * 32]

    return pl.pallas_call(
        body,
        grid=(n // _BN,),
        in_specs=[pl.BlockSpec((_BN, k), lambda i: (i, 0)),
                  pl.BlockSpec((k, wo), lambda i: (0, 0))],
        out_specs=pl.BlockSpec((jc, _BN, 32), lambda i: (0, i, 0)),
        out_shape=jax.ShapeDtypeStruct((jc, n, 32), jnp.float32),
    )(x, w)


def _tc_mm_from_blocked(hb, w):
    n = hb.shape[1]
    wo = w.shape[1]
    jc = wo // 32

    def body(h_ref, w_ref, o_ref):
        h = jnp.concatenate([h_ref[t] for t in range(4)], axis=1)
        y = jnp.dot(h, w_ref[...], preferred_element_type=jnp.float32)
        for t in range(jc):
            o_ref[t] = y[:, t * 32:(t + 1) * 32]

    return pl.pallas_call(
        body,
        grid=(n // _BN,),
        in_specs=[pl.BlockSpec((4, _BN, 32), lambda i: (0, i, 0)),
                  pl.BlockSpec((128, wo), lambda i: (0, 0))],
        out_specs=pl.BlockSpec((jc, _BN, 32), lambda i: (0, i, 0)),
        out_shape=jax.ShapeDtypeStruct((jc, n, 32), jnp.float32),
    )(hb, w)


def _inv_cnt(c0, c1):
    return 1.0 / jnp.maximum(c0[:, 0:1] + c1[:, 0:1], 1.0)


def _tc_combine1_dual(agg_a, ca0, ca1, agg_b, cb0, cb1, x, wa, wb, ba, bb):
    n = x.shape[0]

    def body(aa, a0, a1, ab, b0, b1, x_ref, wa_ref, wb_ref, ba_ref, bb_ref,
             o_ref):
        sa = jnp.concatenate([aa[t] for t in range(4)], axis=1)
        sb = jnp.concatenate([ab[t] for t in range(4)], axis=1)
        y = jnp.dot(x_ref[...], wa_ref[...] + wb_ref[...],
                    preferred_element_type=jnp.float32)
        res = (sa * _inv_cnt(a0, a1) + sb * _inv_cnt(b0, b1) + y
               + ba_ref[...] + bb_ref[...])
        res = jnp.maximum(res, 0.0)
        for t in range(4):
            o_ref[t] = res[:, t * 32:(t + 1) * 32]

    cspec = pl.BlockSpec((_BN, 16), lambda i: (i, 0))
    aspec = pl.BlockSpec((4, _BN, 32), lambda i: (0, i, 0))
    wspec = pl.BlockSpec((128, 128), lambda i: (0, 0))
    bspec = pl.BlockSpec((1, 128), lambda i: (0, 0))
    return pl.pallas_call(
        body,
        grid=(n // _BN,),
        in_specs=[aspec, cspec, cspec, aspec, cspec, cspec,
                  pl.BlockSpec((_BN, 128), lambda i: (i, 0)),
                  wspec, wspec, bspec, bspec],
        out_specs=pl.BlockSpec((4, _BN, 32), lambda i: (0, i, 0)),
        out_shape=jax.ShapeDtypeStruct((4, n, 32), jnp.float32),
    )(agg_a, ca0, ca1, agg_b, cb0, cb1, x, wa, wb, ba, bb)


def _tc_combine1_single(agg_a, ca0, ca1, x, wa, ba):
    n = x.shape[0]

    def body(aa, a0, a1, x_ref, wa_ref, ba_ref, o_ref):
        sa = jnp.concatenate([aa[t] for t in range(4)], axis=1)
        y = jnp.dot(x_ref[...], wa_ref[...],
                    preferred_element_type=jnp.float32)
        res = jnp.maximum(sa * _inv_cnt(a0, a1) + y + ba_ref[...], 0.0)
        for t in range(4):
            o_ref[t] = res[:, t * 32:(t + 1) * 32]

    cspec = pl.BlockSpec((_BN, 16), lambda i: (i, 0))
    return pl.pallas_call(
        body,
        grid=(n // _BN,),
        in_specs=[pl.BlockSpec((4, _BN, 32), lambda i: (0, i, 0)),
                  cspec, cspec,
                  pl.BlockSpec((_BN, 128), lambda i: (i, 0)),
                  pl.BlockSpec((128, 128), lambda i: (0, 0)),
                  pl.BlockSpec((1, 128), lambda i: (0, 0))],
        out_specs=pl.BlockSpec((4, _BN, 32), lambda i: (0, i, 0)),
        out_shape=jax.ShapeDtypeStruct((4, n, 32), jnp.float32),
    )(agg_a, ca0, ca1, x, wa, ba)


def _tc_combine2_dual(agg_a, ca0, ca1, agg_b, cb0, cb1, hb, wa, wb, ba, bb):
    n = hb.shape[1]

    def body(aa, a0, a1, ab, b0, b1, h_ref, wa_ref, wb_ref, ba_ref, bb_ref,
             o_ref):
        sa = jnp.concatenate([aa[0], aa[1]], axis=1)
        sb = jnp.concatenate([ab[0], ab[1]], axis=1)
        h = jnp.concatenate([h_ref[t] for t in range(4)], axis=1)
        y = jnp.dot(h, wa_ref[...] + wb_ref[...],
                    preferred_element_type=jnp.float32)
        o_ref[...] = (sa * _inv_cnt(a0, a1) + sb * _inv_cnt(b0, b1) + y
                      + ba_ref[...] + bb_ref[...])

    cspec = pl.BlockSpec((_BN, 16), lambda i: (i, 0))
    aspec = pl.BlockSpec((2, _BN, 32), lambda i: (0, i, 0))
    wspec = pl.BlockSpec((128, 64), lambda i: (0, 0))
    bspec = pl.BlockSpec((1, 64), lambda i: (0, 0))
    return pl.pallas_call(
        body,
        grid=(n // _BN,),
        in_specs=[aspec, cspec, cspec, aspec, cspec, cspec,
                  pl.BlockSpec((4, _BN, 32), lambda i: (0, i, 0)),
                  wspec, wspec, bspec, bspec],
        out_specs=pl.BlockSpec((_BN, 64), lambda i: (i, 0)),
        out_shape=jax.ShapeDtypeStruct((n, 64), jnp.float32),
    )(agg_a, ca0, ca1, agg_b, cb0, cb1, hb, wa, wb, ba, bb)


def _tc_combine2_single(agg_a, ca0, ca1, hb, wa, ba):
    n = hb.shape[1]

    def body(aa, a0, a1, h_ref, wa_ref, ba_ref, o_ref):
        sa = jnp.concatenate([aa[0], aa[1]], axis=1)
        h = jnp.concatenate([h_ref[t] for t in range(4)], axis=1)
        y = jnp.dot(h, wa_ref[...], preferred_element_type=jnp.float32)
        o_ref[...] = sa * _inv_cnt(a0, a1) + y + ba_ref[...]

    cspec = pl.BlockSpec((_BN, 16), lambda i: (i, 0))
    return pl.pallas_call(
        body,
        grid=(n // _BN,),
        in_specs=[pl.BlockSpec((2, _BN, 32), lambda i: (0, i, 0)),
                  cspec, cspec,
                  pl.BlockSpec((4, _BN, 32), lambda i: (0, i, 0)),
                  pl.BlockSpec((128, 64), lambda i: (0, 0)),
                  pl.BlockSpec((1, 64), lambda i: (0, 0))],
        out_specs=pl.BlockSpec((_BN, 64), lambda i: (i, 0)),
        out_shape=jax.ShapeDtypeStruct((n, 64), jnp.float32),
    )(agg_a, ca0, ca1, hb, wa, ba)


def kernel(x_user, x_product, ei_reviews, ei_rev_reviews, ei_similar,
           w1_rp_l, b1_rp, w1_rp_r, w1_pu_l, b1_pu, w1_pu_r,
           w1_pp_l, b1_pp, w1_pp_r,
           w2_rp_l, b2_rp, w2_rp_r, w2_pu_l, b2_pu, w2_pu_r,
           w2_pp_l, b2_pp, w2_pp_r):
    s_r, d_r = _prep(ei_reviews[0], ei_reviews[1])
    s_v, d_v = _prep(ei_rev_reviews[0], ei_rev_reviews[1])
    s_s, d_s = _prep(ei_similar[0], ei_similar[1])

    cparts = _sc_counts(d_r, d_v, d_s)
    cr0, cr1 = cparts[0, 0], cparts[1, 0]
    cv0, cv1 = cparts[0, 1], cparts[1, 1]
    cs0, cs1 = cparts[0, 2], cparts[1, 2]

    yu1 = _tc_mm_blocked(x_user, w1_rp_l)
    ypp1 = _tc_mm_blocked(x_product, w1_pp_l)
    ypu1 = _tc_mm_blocked(x_product, w1_pu_l)
    agg_r1 = _sc_agg(yu1.reshape(-1, 32), _adj(s_r, 4, _NU), d_r, 4, _NP)
    agg_s1 = _sc_agg(ypp1.reshape(-1, 32), _adj(s_s, 4, _NP), d_s, 4, _NP)
    agg_v1 = _sc_agg(ypu1.reshape(-1, 32), _adj(s_v, 4, _NP), d_v, 4, _NU)
    h_p = _tc_combine1_dual(agg_r1, cr0, cr1, agg_s1, cs0, cs1, x_product,
                            w1_rp_r, w1_pp_r,
                            b1_rp.reshape(1, -1), b1_pp.reshape(1, -1))
    h_u = _tc_combine1_single(agg_v1, cv0, cv1, x_user, w1_pu_r,
                              b1_pu.reshape(1, -1))

    y2r = _tc_mm_from_blocked(h_u, w2_rp_l)
    y2s = _tc_mm_from_blocked(h_p, w2_pp_l)
    y2v = _tc_mm_from_blocked(h_p, w2_pu_l)
    agg_r2 = _sc_agg(y2r.reshape(-1, 32), _adj(s_r, 2, _NU), d_r, 2, _NP)
    agg_s2 = _sc_agg(y2s.reshape(-1, 32), _adj(s_s, 2, _NP), d_s, 2, _NP)
    agg_v2 = _sc_agg(y2v.reshape(-1, 32), _adj(s_v, 2, _NP), d_v, 2, _NU)
    out_p = _tc_combine2_dual(agg_r2, cr0, cr1, agg_s2, cs0, cs1, h_p,
                              w2_rp_r, w2_pp_r,
                              b2_rp.reshape(1, -1), b2_pp.reshape(1, -1))
    out_u = _tc_combine2_single(agg_v2, cv0, cv1, h_u, w2_pu_r,
                                b2_pu.reshape(1, -1))
    return (out_u, out_p)


# speedup vs baseline: 2.6165x; 1.0359x over previous
import jax
import jax.numpy as jnp
from jax import lax
from jax.experimental import pallas as pl
from jax.experimental.pallas import tpu as pltpu
from jax.experimental.pallas import tpu_sc as plsc

_NU = 50000
_NP = 50000
_BN = 2000
_NBIG = 16
_NSUB = 16
_NACC = 50048
_TRASH = 50000
_WPS = 3128
_ZPS = _NACC // _NSUB


def _zero_acc(acc, zbuf, sid):
    def zacc(t, carry):
        pltpu.sync_copy(zbuf, acc.at[pl.ds(sid * _ZPS + t * 128, 128)])
        return carry
    lax.fori_loop(0, _ZPS // 128, zacc, 0)
    rem = _ZPS % 128
    pltpu.sync_copy(zbuf.at[pl.ds(0, rem)],
                    acc.at[pl.ds(sid * _ZPS + _ZPS - rem, rem)])


def _writeout(acc, out_ref, sid, out_base, n_dst):
    last = n_dst - _WPS

    @pl.when(sid < _NSUB - 1)
    def _():
        b = pl.multiple_of(sid * _WPS, 8)
        pltpu.sync_copy(acc.at[pl.ds(b, _WPS)],
                        out_ref.at[pl.ds(pl.multiple_of(out_base + b, 8),
                                         _WPS)])

    @pl.when(sid == _NSUB - 1)
    def _():
        pltpu.sync_copy(acc.at[pl.ds(last, _WPS)],
                        out_ref.at[pl.ds(pl.multiple_of(out_base + last, 8),
                                         _WPS)])


def _prep(src, dst):
    e = src.shape[0]
    nb = ((e + 32767) // 32768) * 256
    ep = nb * 128
    src = jnp.concatenate([src.astype(jnp.int32),
                           jnp.zeros((ep - e,), jnp.int32)])
    dst = jnp.concatenate([dst.astype(jnp.int32),
                           jnp.full((ep - e,), _TRASH, jnp.int32)])
    return src.reshape(nb, 128), dst.reshape(nb, 128)


def _adj(src2d, j_count, n_src):
    off = (jnp.arange(j_count, dtype=jnp.int32) * n_src)[:, None, None]
    return (src2d[None] + off).reshape(j_count * src2d.shape[0], 128)


_MESH = dict(core_axis_name="c", subcore_axis_name="s")


def _sc_agg(table, srcadj, dst2d, j_count, n_dst):
    nb = dst2d.shape[0]
    gps = nb // _NSUB
    nchunk = gps // _NBIG
    rounds = j_count // 2

    def body(table_ref, src_ref, dst_ref, out_ref, acc, srcbig, dstbig,
             r0, r1, r2, r3, g0, g1, g2, g3, s0, s1, s2, s3):
        rows = (r0, r1, r2, r3)
        gsems = (g0, g1, g2, g3)
        ssems = (s0, s1, s2, s3)
        cid = lax.axis_index("c")
        sid = lax.axis_index("s")

        for r in range(rounds):
            j = cid + 2 * r

            def zrow(i, carry):
                r0[i, pl.ds(0, 16)] = jnp.zeros((16,), jnp.float32)
                r0[i, pl.ds(16, 16)] = jnp.zeros((16,), jnp.float32)
                return carry
            lax.fori_loop(0, 128, zrow, 0)
            _zero_acc(acc, r0, sid)
            plsc.subcore_barrier()

            def chunk(cq, carry):
                base = sid * gps + cq * _NBIG
                pltpu.sync_copy(src_ref.at[pl.ds(j * nb + base, _NBIG)],
                                srcbig)
                pltpu.sync_copy(dst_ref.at[pl.ds(base, _NBIG)], dstbig)

                def sstep(ss, carry2):
                    for i in range(4):
                        k = 4 * ss + i

                        @pl.when(ss > 0)
                        def _():
                            pltpu.make_async_copy(
                                rows[i], acc.at[dstbig.at[k]],
                                ssems[i]).wait()
                        pltpu.async_copy(table_ref.at[srcbig.at[k]],
                                         rows[i], gsems[i])
                    for i in range(4):
                        k = 4 * ss + i
                        pltpu.make_async_copy(table_ref.at[srcbig.at[k]],
                                              rows[i], gsems[i]).wait()
                        pltpu.async_copy(rows[i], acc.at[dstbig.at[k]],
                                         ssems[i], add=True)
                    return carry2
                lax.fori_loop(0, _NBIG // 4, sstep, 0)
                for i in range(4):
                    pltpu.make_async_copy(rows[i], acc.at[dstbig.at[i]],
                                          ssems[i]).wait()
                return carry
            lax.fori_loop(0, nchunk, chunk, 0)
            plsc.subcore_barrier()

            _writeout(acc, out_ref, sid, j * n_dst, n_dst)
            plsc.subcore_barrier()

    f = pl.kernel(
        body,
        out_type=jax.ShapeDtypeStruct((j_count * n_dst, 32), jnp.float32),
        mesh=plsc.VectorSubcoreMesh(**_MESH),
        compiler_params=pltpu.CompilerParams(use_tc_tiling_on_sc=False),
        scratch_types=(
            [pltpu.VMEM_SHARED((_NACC, 32), jnp.float32),
             pltpu.VMEM((_NBIG, 128), jnp.int32),
             pltpu.VMEM((_NBIG, 128), jnp.int32)]
            + [pltpu.VMEM((128, 32), jnp.float32) for _ in range(4)]
            + [pltpu.SemaphoreType.DMA] * 8
        ),
    )
    return f(table, srcadj, dst2d).reshape(j_count, n_dst, 32)


def _sc_counts(dst_r, dst_v, dst_s):
    n = _NP

    def body(dr_ref, dv_ref, ds_ref, out_ref, cacc, dstbig, ones, ssem):
        cid = lax.axis_index("c")
        sid = lax.axis_index("s")
        wid = cid * _NSUB + sid

        for t, dref in ((0, dr_ref), (1, dv_ref), (2, ds_ref)):
            nb = dref.shape[0]
            gps = nb // (2 * _NSUB)
            nchunk = gps // _NBIG

            def zrow(i, carry):
                ones[i, pl.ds(0, 16)] = jnp.zeros((16,), jnp.float32)
                return carry
            lax.fori_loop(0, 128, zrow, 0)
            _zero_acc(cacc, ones, sid)

            def frow(i, carry):
                ones[i, pl.ds(0, 16)] = jnp.ones((16,), jnp.float32)
                return carry
            lax.fori_loop(0, 128, frow, 0)
            plsc.subcore_barrier()

            def chunk(cq, carry):
                gb = wid * gps + cq * _NBIG
                pltpu.sync_copy(dref.at[pl.ds(gb, _NBIG)], dstbig)

                def quad(p, carry2):
                    for i in range(4):
                        pltpu.async_copy(ones, cacc.at[dstbig.at[4 * p + i]],
                                         ssem, add=True)
                    return carry2
                lax.fori_loop(0, _NBIG // 4, quad, 0)
                for _ in range(_NBIG):
                    pltpu.make_async_copy(ones, cacc.at[dstbig.at[0]],
                                          ssem).wait()
                return carry
            lax.fori_loop(0, nchunk, chunk, 0)
            plsc.subcore_barrier()

            _writeout(cacc, out_ref, sid, cid * 3 * n + t * n, n)
            plsc.subcore_barrier()

    f = pl.kernel(
        body,
        out_type=jax.ShapeDtypeStruct((2 * 3 * n, 16), jnp.float32),
        mesh=plsc.VectorSubcoreMesh(**_MESH),
        compiler_params=pltpu.CompilerParams(use_tc_tiling_on_sc=False),
        scratch_types=[
            pltpu.VMEM_SHARED((_NACC, 16), jnp.float32),
            pltpu.VMEM((_NBIG, 128), jnp.int32),
            pltpu.VMEM((128, 16), jnp.float32),
            pltpu.SemaphoreType.DMA,
        ],
    )
    return f(dst_r, dst_v, dst_s).reshape(2, 3, n, 16)


def _tc_mm_blocked(x, w):
    n, k = x.shape
    wo = w.shape[1]
    jc = wo // 32

    def body(x_ref, w_ref, o_ref):
        y = jnp.dot(x_ref[...], w_ref[...], preferred_element_type=jnp.float32)
        for t in range(jc):
            o_ref[t] = y[:, t * 32:(t + 1) * 32]

    return pl.pallas_call(
        body,
        grid=(n // _BN,),
        in_specs=[pl.BlockSpec((_BN, k), lambda i: (i, 0)),
                  pl.BlockSpec((k, wo), lambda i: (0, 0))],
        out_specs=pl.BlockSpec((jc, _BN, 32), lambda i: (0, i, 0)),
        out_shape=jax.ShapeDtypeStruct((jc, n, 32), jnp.float32),
    )(x, w)


def _tc_mm_from_blocked(hb, w):
    n = hb.shape[1]
    wo = w.shape[1]
    jc = wo // 32

    def body(h_ref, w_ref, o_ref):
        h = jnp.concatenate([h_ref[t] for t in range(4)], axis=1)
        y = jnp.dot(h, w_ref[...], preferred_element_type=jnp.float32)
        for t in range(jc):
            o_ref[t] = y[:, t * 32:(t + 1) * 32]

    return pl.pallas_call(
        body,
        grid=(n // _BN,),
        in_specs=[pl.BlockSpec((4, _BN, 32), lambda i: (0, i, 0)),
                  pl.BlockSpec((128, wo), lambda i: (0, 0))],
        out_specs=pl.BlockSpec((jc, _BN, 32), lambda i: (0, i, 0)),
        out_shape=jax.ShapeDtypeStruct((jc, n, 32), jnp.float32),
    )(hb, w)


def _inv_cnt(c0, c1):
    return 1.0 / jnp.maximum(c0[:, 0:1] + c1[:, 0:1], 1.0)


def _tc_combine1_dual(agg_a, ca0, ca1, agg_b, cb0, cb1, x, wa, wb, ba, bb):
    n = x.shape[0]

    def body(aa, a0, a1, ab, b0, b1, x_ref, wa_ref, wb_ref, ba_ref, bb_ref,
             o_ref):
        sa = jnp.concatenate([aa[t] for t in range(4)], axis=1)
        sb = jnp.concatenate([ab[t] for t in range(4)], axis=1)
        y = jnp.dot(x_ref[...], wa_ref[...] + wb_ref[...],
                    preferred_element_type=jnp.float32)
        res = (sa * _inv_cnt(a0, a1) + sb * _inv_cnt(b0, b1) + y
               + ba_ref[...] + bb_ref[...])
        res = jnp.maximum(res, 0.0)
        for t in range(4):
            o_ref[t] = res[:, t * 32:(t + 1) * 32]

    cspec = pl.BlockSpec((_BN, 16), lambda i: (i, 0))
    aspec = pl.BlockSpec((4, _BN, 32), lambda i: (0, i, 0))
    wspec = pl.BlockSpec((128, 128), lambda i: (0, 0))
    bspec = pl.BlockSpec((1, 128), lambda i: (0, 0))
    return pl.pallas_call(
        body,
        grid=(n // _BN,),
        in_specs=[aspec, cspec, cspec, aspec, cspec, cspec,
                  pl.BlockSpec((_BN, 128), lambda i: (i, 0)),
                  wspec, wspec, bspec, bspec],
        out_specs=pl.BlockSpec((4, _BN, 32), lambda i: (0, i, 0)),
        out_shape=jax.ShapeDtypeStruct((4, n, 32), jnp.float32),
    )(agg_a, ca0, ca1, agg_b, cb0, cb1, x, wa, wb, ba, bb)


def _tc_combine1_single(agg_a, ca0, ca1, x, wa, ba):
    n = x.shape[0]

    def body(aa, a0, a1, x_ref, wa_ref, ba_ref, o_ref):
        sa = jnp.concatenate([aa[t] for t in range(4)], axis=1)
        y = jnp.dot(x_ref[...], wa_ref[...],
                    preferred_element_type=jnp.float32)
        res = jnp.maximum(sa * _inv_cnt(a0, a1) + y + ba_ref[...], 0.0)
        for t in range(4):
            o_ref[t] = res[:, t * 32:(t + 1) * 32]

    cspec = pl.BlockSpec((_BN, 16), lambda i: (i, 0))
    return pl.pallas_call(
        body,
        grid=(n // _BN,),
        in_specs=[pl.BlockSpec((4, _BN, 32), lambda i: (0, i, 0)),
                  cspec, cspec,
                  pl.BlockSpec((_BN, 128), lambda i: (i, 0)),
                  pl.BlockSpec((128, 128), lambda i: (0, 0)),
                  pl.BlockSpec((1, 128), lambda i: (0, 0))],
        out_specs=pl.BlockSpec((4, _BN, 32), lambda i: (0, i, 0)),
        out_shape=jax.ShapeDtypeStruct((4, n, 32), jnp.float32),
    )(agg_a, ca0, ca1, x, wa, ba)


def _tc_combine2_dual(agg_a, ca0, ca1, agg_b, cb0, cb1, hb, wa, wb, ba, bb):
    n = hb.shape[1]

    def body(aa, a0, a1, ab, b0, b1, h_ref, wa_ref, wb_ref, ba_ref, bb_ref,
             o_ref):
        sa = jnp.concatenate([aa[0], aa[1]], axis=1)
        sb = jnp.concatenate([ab[0], ab[1]], axis=1)
        h = jnp.concatenate([h_ref[t] for t in range(4)], axis=1)
        y = jnp.dot(h, wa_ref[...] + wb_ref[...],
                    preferred_element_type=jnp.float32)
        o_ref[...] = (sa * _inv_cnt(a0, a1) + sb * _inv_cnt(b0, b1) + y
                      + ba_ref[...] + bb_ref[...])

    cspec = pl.BlockSpec((_BN, 16), lambda i: (i, 0))
    aspec = pl.BlockSpec((2, _BN, 32), lambda i: (0, i, 0))
    wspec = pl.BlockSpec((128, 64), lambda i: (0, 0))
    bspec = pl.BlockSpec((1, 64), lambda i: (0, 0))
    return pl.pallas_call(
        body,
        grid=(n // _BN,),
        in_specs=[aspec, cspec, cspec, aspec, cspec, cspec,
                  pl.BlockSpec((4, _BN, 32), lambda i: (0, i, 0)),
                  wspec, wspec, bspec, bspec],
        out_specs=pl.BlockSpec((_BN, 64), lambda i: (i, 0)),
        out_shape=jax.ShapeDtypeStruct((n, 64), jnp.float32),
    )(agg_a, ca0, ca1, agg_b, cb0, cb1, hb, wa, wb, ba, bb)


def _tc_combine2_single(agg_a, ca0, ca1, hb, wa, ba):
    n = hb.shape[1]

    def body(aa, a0, a1, h_ref, wa_ref, ba_ref, o_ref):
        sa = jnp.concatenate([aa[0], aa[1]], axis=1)
        h = jnp.concatenate([h_ref[t] for t in range(4)], axis=1)
        y = jnp.dot(h, wa_ref[...], preferred_element_type=jnp.float32)
        o_ref[...] = sa * _inv_cnt(a0, a1) + y + ba_ref[...]

    cspec = pl.BlockSpec((_BN, 16), lambda i: (i, 0))
    return pl.pallas_call(
        body,
        grid=(n // _BN,),
        in_specs=[pl.BlockSpec((2, _BN, 32), lambda i: (0, i, 0)),
                  cspec, cspec,
                  pl.BlockSpec((4, _BN, 32), lambda i: (0, i, 0)),
                  pl.BlockSpec((128, 64), lambda i: (0, 0)),
                  pl.BlockSpec((1, 64), lambda i: (0, 0))],
        out_specs=pl.BlockSpec((_BN, 64), lambda i: (i, 0)),
        out_shape=jax.ShapeDtypeStruct((n, 64), jnp.float32),
    )(agg_a, ca0, ca1, hb, wa, ba)


def kernel(x_user, x_product, ei_reviews, ei_rev_reviews, ei_similar,
           w1_rp_l, b1_rp, w1_rp_r, w1_pu_l, b1_pu, w1_pu_r,
           w1_pp_l, b1_pp, w1_pp_r,
           w2_rp_l, b2_rp, w2_rp_r, w2_pu_l, b2_pu, w2_pu_r,
           w2_pp_l, b2_pp, w2_pp_r):
    s_r, d_r = _prep(ei_reviews[0], ei_reviews[1])
    s_v, d_v = _prep(ei_rev_reviews[0], ei_rev_reviews[1])
    s_s, d_s = _prep(ei_similar[0], ei_similar[1])

    cparts = _sc_counts(d_r, d_v, d_s)
    cr0, cr1 = cparts[0, 0], cparts[1, 0]
    cv0, cv1 = cparts[0, 1], cparts[1, 1]
    cs0, cs1 = cparts[0, 2], cparts[1, 2]

    yu1 = _tc_mm_blocked(x_user, w1_rp_l)
    ypp1 = _tc_mm_blocked(x_product, w1_pp_l)
    ypu1 = _tc_mm_blocked(x_product, w1_pu_l)
    agg_r1 = _sc_agg(yu1.reshape(-1, 32), _adj(s_r, 4, _NU), d_r, 4, _NP)
    agg_s1 = _sc_agg(ypp1.reshape(-1, 32), _adj(s_s, 4, _NP), d_s, 4, _NP)
    agg_v1 = _sc_agg(ypu1.reshape(-1, 32), _adj(s_v, 4, _NP), d_v, 4, _NU)
    h_p = _tc_combine1_dual(agg_r1, cr0, cr1, agg_s1, cs0, cs1, x_product,
                            w1_rp_r, w1_pp_r,
                            b1_rp.reshape(1, -1), b1_pp.reshape(1, -1))
    h_u = _tc_combine1_single(agg_v1, cv0, cv1, x_user, w1_pu_r,
                              b1_pu.reshape(1, -1))

    y2r = _tc_mm_from_blocked(h_u, w2_rp_l)
    y2s = _tc_mm_from_blocked(h_p, w2_pp_l)
    y2v = _tc_mm_from_blocked(h_p, w2_pu_l)
    agg_r2 = _sc_agg(y2r.reshape(-1, 32), _adj(s_r, 2, _NU), d_r, 2, _NP)
    agg_s2 = _sc_agg(y2s.reshape(-1, 32), _adj(s_s, 2, _NP), d_s, 2, _NP)
    agg_v2 = _sc_agg(y2v.reshape(-1, 32), _adj(s_v, 2, _NP), d_v, 2, _NU)
    out_p = _tc_combine2_dual(agg_r2, cr0, cr1, agg_s2, cs0, cs1, h_p,
                              w2_rp_r, w2_pp_r,
                              b2_rp.reshape(1, -1), b2_pp.reshape(1, -1))
    out_u = _tc_combine2_single(agg_v2, cv0, cv1, h_u, w2_pu_r,
                                b2_pu.reshape(1, -1))
    return (out_u, out_p)


# speedup vs baseline: 2.6616x; 1.0172x over previous
import jax
import jax.numpy as jnp
from jax import lax
from jax.experimental import pallas as pl
from jax.experimental.pallas import tpu as pltpu
from jax.experimental.pallas import tpu_sc as plsc

_NU = 50000
_NP = 50000
_BN = 2000
_NBIG = 16
_NSUB = 16
_NACC = 50048
_TRASH = 50000
_WPS = 3128
_ZPS = _NACC // _NSUB


def _zero_acc(acc, zbuf, sid):
    def zacc(t, carry):
        pltpu.sync_copy(zbuf, acc.at[pl.ds(sid * _ZPS + t * 128, 128)])
        return carry
    lax.fori_loop(0, _ZPS // 128, zacc, 0)
    rem = _ZPS % 128
    pltpu.sync_copy(zbuf.at[pl.ds(0, rem)],
                    acc.at[pl.ds(sid * _ZPS + _ZPS - rem, rem)])


def _writeout(acc, out_ref, sid, out_base, n_dst):
    last = n_dst - _WPS

    @pl.when(sid < _NSUB - 1)
    def _():
        b = pl.multiple_of(sid * _WPS, 8)
        pltpu.sync_copy(acc.at[pl.ds(b, _WPS)],
                        out_ref.at[pl.ds(pl.multiple_of(out_base + b, 8),
                                         _WPS)])

    @pl.when(sid == _NSUB - 1)
    def _():
        pltpu.sync_copy(acc.at[pl.ds(last, _WPS)],
                        out_ref.at[pl.ds(pl.multiple_of(out_base + last, 8),
                                         _WPS)])


def _prep(src, dst):
    e = src.shape[0]
    nb = ((e + 32767) // 32768) * 256
    ep = nb * 128
    src = jnp.concatenate([src.astype(jnp.int32),
                           jnp.zeros((ep - e,), jnp.int32)])
    dst = jnp.concatenate([dst.astype(jnp.int32),
                           jnp.full((ep - e,), _TRASH, jnp.int32)])
    return src.reshape(nb, 128), dst.reshape(nb, 128)


def _adj(src2d, j_count, n_src):
    off = (jnp.arange(j_count, dtype=jnp.int32) * n_src)[:, None, None]
    return (src2d[None] + off).reshape(j_count * src2d.shape[0], 128)


_MESH = dict(core_axis_name="c", subcore_axis_name="s")


def _sc_agg(table, srcadj, dst2d, j_count, n_dst):
    nb = dst2d.shape[0]
    gps = nb // _NSUB
    nchunk = gps // _NBIG
    rounds = j_count // 2

    def body(table_ref, src_ref, dst_ref, out_ref, acc, sb0, db0, sb1, db1,
             r0, r1, r2, r3, g0, g1, g2, g3, s0, s1, s2, s3, esem):
        rows = (r0, r1, r2, r3)
        gsems = (g0, g1, g2, g3)
        ssems = (s0, s1, s2, s3)
        ebufs = ((sb0, db0), (sb1, db1))
        cid = lax.axis_index("c")
        sid = lax.axis_index("s")

        def load_chunk(j, cq, par):
            @pl.when(cq < nchunk)
            def _():
                base = sid * gps + cq * _NBIG
                pltpu.async_copy(src_ref.at[pl.ds(j * nb + base, _NBIG)],
                                 ebufs[par][0], esem)
                pltpu.async_copy(dst_ref.at[pl.ds(base, _NBIG)],
                                 ebufs[par][1], esem)

        def wait_chunk(par):
            pltpu.make_async_copy(src_ref.at[pl.ds(0, _NBIG)],
                                  ebufs[par][0], esem).wait()
            pltpu.make_async_copy(dst_ref.at[pl.ds(0, _NBIG)],
                                  ebufs[par][1], esem).wait()

        for r in range(rounds):
            j = cid + 2 * r

            def zrow(i, carry):
                r0[i, pl.ds(0, 16)] = jnp.zeros((16,), jnp.float32)
                r0[i, pl.ds(16, 16)] = jnp.zeros((16,), jnp.float32)
                return carry
            lax.fori_loop(0, 128, zrow, 0)
            _zero_acc(acc, r0, sid)
            plsc.subcore_barrier()

            load_chunk(j, 0, 0)

            def chunk_pair(q, carry):
                for par in range(2):
                    cq = 2 * q + par
                    srcbig, dstbig = ebufs[par]
                    wait_chunk(par)

                    def sstep(ss, carry2):
                        for i in range(4):
                            k = 4 * ss + i

                            @pl.when((ss > 0) | (cq > 0))
                            def _():
                                pltpu.make_async_copy(
                                    rows[i], acc.at[dstbig.at[k]],
                                    ssems[i]).wait()
                            pltpu.async_copy(table_ref.at[srcbig.at[k]],
                                             rows[i], gsems[i])
                        for i in range(4):
                            k = 4 * ss + i
                            pltpu.make_async_copy(
                                table_ref.at[srcbig.at[k]], rows[i],
                                gsems[i]).wait()
                            pltpu.async_copy(rows[i], acc.at[dstbig.at[k]],
                                             ssems[i], add=True)
                        return carry2
                    sstep(0, 0)
                    load_chunk(j, cq + 1, 1 - par)
                    lax.fori_loop(1, _NBIG // 4, sstep, 0)
                return carry
            lax.fori_loop(0, nchunk // 2, chunk_pair, 0)
            for i in range(4):
                pltpu.make_async_copy(rows[i], acc.at[db1.at[i]],
                                      ssems[i]).wait()
            plsc.subcore_barrier()

            _writeout(acc, out_ref, sid, j * n_dst, n_dst)
            plsc.subcore_barrier()

    f = pl.kernel(
        body,
        out_type=jax.ShapeDtypeStruct((j_count * n_dst, 32), jnp.float32),
        mesh=plsc.VectorSubcoreMesh(**_MESH),
        compiler_params=pltpu.CompilerParams(use_tc_tiling_on_sc=False),
        scratch_types=(
            [pltpu.VMEM_SHARED((_NACC, 32), jnp.float32)]
            + [pltpu.VMEM((_NBIG, 128), jnp.int32) for _ in range(4)]
            + [pltpu.VMEM((128, 32), jnp.float32) for _ in range(4)]
            + [pltpu.SemaphoreType.DMA] * 9
        ),
    )
    return f(table, srcadj, dst2d).reshape(j_count, n_dst, 32)


def _sc_counts(dst_r, dst_v, dst_s):
    n = _NP

    def body(dr_ref, dv_ref, ds_ref, out_ref, cacc, dstbig, ones, ssem):
        cid = lax.axis_index("c")
        sid = lax.axis_index("s")
        wid = cid * _NSUB + sid

        for t, dref in ((0, dr_ref), (1, dv_ref), (2, ds_ref)):
            nb = dref.shape[0]
            gps = nb // (2 * _NSUB)
            nchunk = gps // _NBIG

            def zrow(i, carry):
                ones[i, pl.ds(0, 16)] = jnp.zeros((16,), jnp.float32)
                return carry
            lax.fori_loop(0, 128, zrow, 0)
            _zero_acc(cacc, ones, sid)

            def frow(i, carry):
                ones[i, pl.ds(0, 16)] = jnp.ones((16,), jnp.float32)
                return carry
            lax.fori_loop(0, 128, frow, 0)
            plsc.subcore_barrier()

            def chunk(cq, carry):
                gb = wid * gps + cq * _NBIG
                pltpu.sync_copy(dref.at[pl.ds(gb, _NBIG)], dstbig)

                def quad(p, carry2):
                    for i in range(4):
                        pltpu.async_copy(ones, cacc.at[dstbig.at[4 * p + i]],
                                         ssem, add=True)
                    return carry2
                lax.fori_loop(0, _NBIG // 4, quad, 0)
                for _ in range(_NBIG):
                    pltpu.make_async_copy(ones, cacc.at[dstbig.at[0]],
                                          ssem).wait()
                return carry
            lax.fori_loop(0, nchunk, chunk, 0)
            plsc.subcore_barrier()

            _writeout(cacc, out_ref, sid, cid * 3 * n + t * n, n)
            plsc.subcore_barrier()

    f = pl.kernel(
        body,
        out_type=jax.ShapeDtypeStruct((2 * 3 * n, 16), jnp.float32),
        mesh=plsc.VectorSubcoreMesh(**_MESH),
        compiler_params=pltpu.CompilerParams(use_tc_tiling_on_sc=False),
        scratch_types=[
            pltpu.VMEM_SHARED((_NACC, 16), jnp.float32),
            pltpu.VMEM((_NBIG, 128), jnp.int32),
            pltpu.VMEM((128, 16), jnp.float32),
            pltpu.SemaphoreType.DMA,
        ],
    )
    return f(dst_r, dst_v, dst_s).reshape(2, 3, n, 16)


def _tc_mm_blocked(x, w):
    n, k = x.shape
    wo = w.shape[1]
    jc = wo // 32

    def body(x_ref, w_ref, o_ref):
        y = jnp.dot(x_ref[...], w_ref[...], preferred_element_type=jnp.float32)
        for t in range(jc):
            o_ref[t] = y[:, t * 32:(t + 1) * 32]

    return pl.pallas_call(
        body,
        grid=(n // _BN,),
        in_specs=[pl.BlockSpec((_BN, k), lambda i: (i, 0)),
                  pl.BlockSpec((k, wo), lambda i: (0, 0))],
        out_specs=pl.BlockSpec((jc, _BN, 32), lambda i: (0, i, 0)),
        out_shape=jax.ShapeDtypeStruct((jc, n, 32), jnp.float32),
    )(x, w)


def _tc_mm_from_blocked(hb, w):
    n = hb.shape[1]
    wo = w.shape[1]
    jc = wo // 32

    def body(h_ref, w_ref, o_ref):
        h = jnp.concatenate([h_ref[t] for t in range(4)], axis=1)
        y = jnp.dot(h, w_ref[...], preferred_element_type=jnp.float32)
        for t in range(jc):
            o_ref[t] = y[:, t * 32:(t + 1) * 32]

    return pl.pallas_call(
        body,
        grid=(n // _BN,),
        in_specs=[pl.BlockSpec((4, _BN, 32), lambda i: (0, i, 0)),
                  pl.BlockSpec((128, wo), lambda i: (0, 0))],
        out_specs=pl.BlockSpec((jc, _BN, 32), lambda i: (0, i, 0)),
        out_shape=jax.ShapeDtypeStruct((jc, n, 32), jnp.float32),
    )(hb, w)


def _inv_cnt(c0, c1):
    return 1.0 / jnp.maximum(c0[:, 0:1] + c1[:, 0:1], 1.0)


def _tc_combine1_dual(agg_a, ca0, ca1, agg_b, cb0, cb1, x, wa, wb, ba, bb):
    n = x.shape[0]

    def body(aa, a0, a1, ab, b0, b1, x_ref, wa_ref, wb_ref, ba_ref, bb_ref,
             o_ref):
        sa = jnp.concatenate([aa[t] for t in range(4)], axis=1)
        sb = jnp.concatenate([ab[t] for t in range(4)], axis=1)
        y = jnp.dot(x_ref[...], wa_ref[...] + wb_ref[...],
                    preferred_element_type=jnp.float32)
        res = (sa * _inv_cnt(a0, a1) + sb * _inv_cnt(b0, b1) + y
               + ba_ref[...] + bb_ref[...])
        res = jnp.maximum(res, 0.0)
        for t in range(4):
            o_ref[t] = res[:, t * 32:(t + 1) * 32]

    cspec = pl.BlockSpec((_BN, 16), lambda i: (i, 0))
    aspec = pl.BlockSpec((4, _BN, 32), lambda i: (0, i, 0))
    wspec = pl.BlockSpec((128, 128), lambda i: (0, 0))
    bspec = pl.BlockSpec((1, 128), lambda i: (0, 0))
    return pl.pallas_call(
        body,
        grid=(n // _BN,),
        in_specs=[aspec, cspec, cspec, aspec, cspec, cspec,
                  pl.BlockSpec((_BN, 128), lambda i: (i, 0)),
                  wspec, wspec, bspec, bspec],
        out_specs=pl.BlockSpec((4, _BN, 32), lambda i: (0, i, 0)),
        out_shape=jax.ShapeDtypeStruct((4, n, 32), jnp.float32),
    )(agg_a, ca0, ca1, agg_b, cb0, cb1, x, wa, wb, ba, bb)


def _tc_combine1_single(agg_a, ca0, ca1, x, wa, ba):
    n = x.shape[0]

    def body(aa, a0, a1, x_ref, wa_ref, ba_ref, o_ref):
        sa = jnp.concatenate([aa[t] for t in range(4)], axis=1)
        y = jnp.dot(x_ref[...], wa_ref[...],
                    preferred_element_type=jnp.float32)
        res = jnp.maximum(sa * _inv_cnt(a0, a1) + y + ba_ref[...], 0.0)
        for t in range(4):
            o_ref[t] = res[:, t * 32:(t + 1) * 32]

    cspec = pl.BlockSpec((_BN, 16), lambda i: (i, 0))
    return pl.pallas_call(
        body,
        grid=(n // _BN,),
        in_specs=[pl.BlockSpec((4, _BN, 32), lambda i: (0, i, 0)),
                  cspec, cspec,
                  pl.BlockSpec((_BN, 128), lambda i: (i, 0)),
                  pl.BlockSpec((128, 128), lambda i: (0, 0)),
                  pl.BlockSpec((1, 128), lambda i: (0, 0))],
        out_specs=pl.BlockSpec((4, _BN, 32), lambda i: (0, i, 0)),
        out_shape=jax.ShapeDtypeStruct((4, n, 32), jnp.float32),
    )(agg_a, ca0, ca1, x, wa, ba)


def _tc_combine2_dual(agg_a, ca0, ca1, agg_b, cb0, cb1, hb, wa, wb, ba, bb):
    n = hb.shape[1]

    def body(aa, a0, a1, ab, b0, b1, h_ref, wa_ref, wb_ref, ba_ref, bb_ref,
             o_ref):
        sa = jnp.concatenate([aa[0], aa[1]], axis=1)
        sb = jnp.concatenate([ab[0], ab[1]], axis=1)
        h = jnp.concatenate([h_ref[t] for t in range(4)], axis=1)
        y = jnp.dot(h, wa_ref[...] + wb_ref[...],
                    preferred_element_type=jnp.float32)
        o_ref[...] = (sa * _inv_cnt(a0, a1) + sb * _inv_cnt(b0, b1) + y
                      + ba_ref[...] + bb_ref[...])

    cspec = pl.BlockSpec((_BN, 16), lambda i: (i, 0))
    aspec = pl.BlockSpec((2, _BN, 32), lambda i: (0, i, 0))
    wspec = pl.BlockSpec((128, 64), lambda i: (0, 0))
    bspec = pl.BlockSpec((1, 64), lambda i: (0, 0))
    return pl.pallas_call(
        body,
        grid=(n // _BN,),
        in_specs=[aspec, cspec, cspec, aspec, cspec, cspec,
                  pl.BlockSpec((4, _BN, 32), lambda i: (0, i, 0)),
                  wspec, wspec, bspec, bspec],
        out_specs=pl.BlockSpec((_BN, 64), lambda i: (i, 0)),
        out_shape=jax.ShapeDtypeStruct((n, 64), jnp.float32),
    )(agg_a, ca0, ca1, agg_b, cb0, cb1, hb, wa, wb, ba, bb)


def _tc_combine2_single(agg_a, ca0, ca1, hb, wa, ba):
    n = hb.shape[1]

    def body(aa, a0, a1, h_ref, wa_ref, ba_ref, o_ref):
        sa = jnp.concatenate([aa[0], aa[1]], axis=1)
        h = jnp.concatenate([h_ref[t] for t in range(4)], axis=1)
        y = jnp.dot(h, wa_ref[...], preferred_element_type=jnp.float32)
        o_ref[...] = sa * _inv_cnt(a0, a1) + y + ba_ref[...]

    cspec = pl.BlockSpec((_BN, 16), lambda i: (i, 0))
    return pl.pallas_call(
        body,
        grid=(n // _BN,),
        in_specs=[pl.BlockSpec((2, _BN, 32), lambda i: (0, i, 0)),
                  cspec, cspec,
                  pl.BlockSpec((4, _BN, 32), lambda i: (0, i, 0)),
                  pl.BlockSpec((128, 64), lambda i: (0, 0)),
                  pl.BlockSpec((1, 64), lambda i: (0, 0))],
        out_specs=pl.BlockSpec((_BN, 64), lambda i: (i, 0)),
        out_shape=jax.ShapeDtypeStruct((n, 64), jnp.float32),
    )(agg_a, ca0, ca1, hb, wa, ba)


def kernel(x_user, x_product, ei_reviews, ei_rev_reviews, ei_similar,
           w1_rp_l, b1_rp, w1_rp_r, w1_pu_l, b1_pu, w1_pu_r,
           w1_pp_l, b1_pp, w1_pp_r,
           w2_rp_l, b2_rp, w2_rp_r, w2_pu_l, b2_pu, w2_pu_r,
           w2_pp_l, b2_pp, w2_pp_r):
    s_r, d_r = _prep(ei_reviews[0], ei_reviews[1])
    s_v, d_v = _prep(ei_rev_reviews[0], ei_rev_reviews[1])
    s_s, d_s = _prep(ei_similar[0], ei_similar[1])

    cparts = _sc_counts(d_r, d_v, d_s)
    cr0, cr1 = cparts[0, 0], cparts[1, 0]
    cv0, cv1 = cparts[0, 1], cparts[1, 1]
    cs0, cs1 = cparts[0, 2], cparts[1, 2]

    yu1 = _tc_mm_blocked(x_user, w1_rp_l)
    ypp1 = _tc_mm_blocked(x_product, w1_pp_l)
    ypu1 = _tc_mm_blocked(x_product, w1_pu_l)
    agg_r1 = _sc_agg(yu1.reshape(-1, 32), _adj(s_r, 4, _NU), d_r, 4, _NP)
    agg_s1 = _sc_agg(ypp1.reshape(-1, 32), _adj(s_s, 4, _NP), d_s, 4, _NP)
    agg_v1 = _sc_agg(ypu1.reshape(-1, 32), _adj(s_v, 4, _NP), d_v, 4, _NU)
    h_p = _tc_combine1_dual(agg_r1, cr0, cr1, agg_s1, cs0, cs1, x_product,
                            w1_rp_r, w1_pp_r,
                            b1_rp.reshape(1, -1), b1_pp.reshape(1, -1))
    h_u = _tc_combine1_single(agg_v1, cv0, cv1, x_user, w1_pu_r,
                              b1_pu.reshape(1, -1))

    y2r = _tc_mm_from_blocked(h_u, w2_rp_l)
    y2s = _tc_mm_from_blocked(h_p, w2_pp_l)
    y2v = _tc_mm_from_blocked(h_p, w2_pu_l)
    agg_r2 = _sc_agg(y2r.reshape(-1, 32), _adj(s_r, 2, _NU), d_r, 2, _NP)
    agg_s2 = _sc_agg(y2s.reshape(-1, 32), _adj(s_s, 2, _NP), d_s, 2, _NP)
    agg_v2 = _sc_agg(y2v.reshape(-1, 32), _adj(s_v, 2, _NP), d_v, 2, _NU)
    out_p = _tc_combine2_dual(agg_r2, cr0, cr1, agg_s2, cs0, cs1, h_p,
                              w2_rp_r, w2_pp_r,
                              b2_rp.reshape(1, -1), b2_pp.reshape(1, -1))
    out_u = _tc_combine2_single(agg_v2, cv0, cv1, h_u, w2_pu_r,
                                b2_pu.reshape(1, -1))
    return (out_u, out_p)


# speedup vs baseline: 2.6735x; 1.0045x over previous
import jax
import jax.numpy as jnp
from jax import lax
from jax.experimental import pallas as pl
from jax.experimental.pallas import tpu as pltpu
from jax.experimental.pallas import tpu_sc as plsc

_NU = 50000
_NP = 50000
_BN = 2000
_NBIG = 16
_NSUB = 16
_NACC = 50048
_TRASH = 50000
_WPS = 3128
_ZPS = _NACC // _NSUB


def _zero_acc(acc, zbuf, sid, zsem):
    def zacc(t, carry):
        pltpu.async_copy(zbuf, acc.at[pl.ds(sid * _ZPS + t * 128, 128)],
                         zsem)
        return carry
    lax.fori_loop(0, _ZPS // 128, zacc, 0)
    rem = _ZPS % 128
    pltpu.async_copy(zbuf.at[pl.ds(0, rem)],
                     acc.at[pl.ds(sid * _ZPS + _ZPS - rem, rem)], zsem)

    def zwait(t, carry):
        pltpu.make_async_copy(
            zbuf, acc.at[pl.ds(sid * _ZPS, 128)], zsem).wait()
        return carry
    lax.fori_loop(0, _ZPS // 128, zwait, 0)
    pltpu.make_async_copy(zbuf.at[pl.ds(0, rem)],
                          acc.at[pl.ds(sid * _ZPS, rem)], zsem).wait()


def _writeout(acc, out_ref, sid, out_base, n_dst):
    last = n_dst - _WPS

    @pl.when(sid < _NSUB - 1)
    def _():
        b = pl.multiple_of(sid * _WPS, 8)
        pltpu.sync_copy(acc.at[pl.ds(b, _WPS)],
                        out_ref.at[pl.ds(pl.multiple_of(out_base + b, 8),
                                         _WPS)])

    @pl.when(sid == _NSUB - 1)
    def _():
        pltpu.sync_copy(acc.at[pl.ds(last, _WPS)],
                        out_ref.at[pl.ds(pl.multiple_of(out_base + last, 8),
                                         _WPS)])


def _prep(src, dst):
    e = src.shape[0]
    nb = ((e + 32767) // 32768) * 256
    ep = nb * 128
    src = jnp.concatenate([src.astype(jnp.int32),
                           jnp.zeros((ep - e,), jnp.int32)])
    dst = jnp.concatenate([dst.astype(jnp.int32),
                           jnp.full((ep - e,), _TRASH, jnp.int32)])
    return src.reshape(nb, 128), dst.reshape(nb, 128)


def _adj(src2d, j_count, n_src):
    off = (jnp.arange(j_count, dtype=jnp.int32) * n_src)[:, None, None]
    return (src2d[None] + off).reshape(j_count * src2d.shape[0], 128)


_MESH = dict(core_axis_name="c", subcore_axis_name="s")


def _sc_agg(table, srcadj, dst2d, j_count, n_dst):
    nb = dst2d.shape[0]
    gps = nb // _NSUB
    nchunk = gps // _NBIG
    rounds = j_count // 2

    def body(table_ref, src_ref, dst_ref, out_ref, acc, sb0, db0, sb1, db1,
             r0, r1, r2, r3, g0, g1, g2, g3, s0, s1, s2, s3, esem):
        rows = (r0, r1, r2, r3)
        gsems = (g0, g1, g2, g3)
        ssems = (s0, s1, s2, s3)
        ebufs = ((sb0, db0), (sb1, db1))
        cid = lax.axis_index("c")
        sid = lax.axis_index("s")

        def load_chunk(j, cq, par):
            @pl.when(cq < nchunk)
            def _():
                base = sid * gps + cq * _NBIG
                pltpu.async_copy(src_ref.at[pl.ds(j * nb + base, _NBIG)],
                                 ebufs[par][0], esem)
                pltpu.async_copy(dst_ref.at[pl.ds(base, _NBIG)],
                                 ebufs[par][1], esem)

        def wait_chunk(par):
            pltpu.make_async_copy(src_ref.at[pl.ds(0, _NBIG)],
                                  ebufs[par][0], esem).wait()
            pltpu.make_async_copy(dst_ref.at[pl.ds(0, _NBIG)],
                                  ebufs[par][1], esem).wait()

        for r in range(rounds):
            j = cid + 2 * r

            def zrow(i, carry):
                r0[i, pl.ds(0, 16)] = jnp.zeros((16,), jnp.float32)
                r0[i, pl.ds(16, 16)] = jnp.zeros((16,), jnp.float32)
                return carry
            lax.fori_loop(0, 128, zrow, 0)
            _zero_acc(acc, r0, sid, g0)
            plsc.subcore_barrier()

            load_chunk(j, 0, 0)

            def chunk_pair(q, carry):
                for par in range(2):
                    cq = 2 * q + par
                    srcbig, dstbig = ebufs[par]
                    wait_chunk(par)

                    def sstep(ss, carry2):
                        for i in range(4):
                            k = 4 * ss + i

                            @pl.when((ss > 0) | (cq > 0))
                            def _():
                                pltpu.make_async_copy(
                                    rows[i], acc.at[dstbig.at[k]],
                                    ssems[i]).wait()
                            pltpu.async_copy(table_ref.at[srcbig.at[k]],
                                             rows[i], gsems[i])
                        for i in range(4):
                            k = 4 * ss + i
                            pltpu.make_async_copy(
                                table_ref.at[srcbig.at[k]], rows[i],
                                gsems[i]).wait()
                            pltpu.async_copy(rows[i], acc.at[dstbig.at[k]],
                                             ssems[i], add=True)
                        return carry2
                    sstep(0, 0)
                    load_chunk(j, cq + 1, 1 - par)
                    lax.fori_loop(1, _NBIG // 4, sstep, 0)
                return carry
            lax.fori_loop(0, nchunk // 2, chunk_pair, 0)
            for i in range(4):
                pltpu.make_async_copy(rows[i], acc.at[db1.at[i]],
                                      ssems[i]).wait()
            plsc.subcore_barrier()

            _writeout(acc, out_ref, sid, j * n_dst, n_dst)
            plsc.subcore_barrier()

    f = pl.kernel(
        body,
        out_type=jax.ShapeDtypeStruct((j_count * n_dst, 32), jnp.float32),
        mesh=plsc.VectorSubcoreMesh(**_MESH),
        compiler_params=pltpu.CompilerParams(use_tc_tiling_on_sc=False),
        scratch_types=(
            [pltpu.VMEM_SHARED((_NACC, 32), jnp.float32)]
            + [pltpu.VMEM((_NBIG, 128), jnp.int32) for _ in range(4)]
            + [pltpu.VMEM((128, 32), jnp.float32) for _ in range(4)]
            + [pltpu.SemaphoreType.DMA] * 9
        ),
    )
    return f(table, srcadj, dst2d).reshape(j_count, n_dst, 32)


def _sc_counts(dst_r, dst_v, dst_s):
    n = _NP

    def body(dr_ref, dv_ref, ds_ref, out_ref, cacc, dstbig, ones, ssem):
        cid = lax.axis_index("c")
        sid = lax.axis_index("s")
        wid = cid * _NSUB + sid

        for t, dref in ((0, dr_ref), (1, dv_ref), (2, ds_ref)):
            nb = dref.shape[0]
            gps = nb // (2 * _NSUB)
            nchunk = gps // _NBIG

            def zrow(i, carry):
                ones[i, pl.ds(0, 16)] = jnp.zeros((16,), jnp.float32)
                return carry
            lax.fori_loop(0, 128, zrow, 0)
            _zero_acc(cacc, ones, sid, ssem)

            def frow(i, carry):
                ones[i, pl.ds(0, 16)] = jnp.ones((16,), jnp.float32)
                return carry
            lax.fori_loop(0, 128, frow, 0)
            plsc.subcore_barrier()

            def chunk(cq, carry):
                gb = wid * gps + cq * _NBIG
                pltpu.sync_copy(dref.at[pl.ds(gb, _NBIG)], dstbig)

                def quad(p, carry2):
                    for i in range(4):
                        pltpu.async_copy(ones, cacc.at[dstbig.at[4 * p + i]],
                                         ssem, add=True)
                    return carry2
                lax.fori_loop(0, _NBIG // 4, quad, 0)
                for _ in range(_NBIG):
                    pltpu.make_async_copy(ones, cacc.at[dstbig.at[0]],
                                          ssem).wait()
                return carry
            lax.fori_loop(0, nchunk, chunk, 0)
            plsc.subcore_barrier()

            _writeout(cacc, out_ref, sid, cid * 3 * n + t * n, n)
            plsc.subcore_barrier()

    f = pl.kernel(
        body,
        out_type=jax.ShapeDtypeStruct((2 * 3 * n, 16), jnp.float32),
        mesh=plsc.VectorSubcoreMesh(**_MESH),
        compiler_params=pltpu.CompilerParams(use_tc_tiling_on_sc=False),
        scratch_types=[
            pltpu.VMEM_SHARED((_NACC, 16), jnp.float32),
            pltpu.VMEM((_NBIG, 128), jnp.int32),
            pltpu.VMEM((128, 16), jnp.float32),
            pltpu.SemaphoreType.DMA,
        ],
    )
    return f(dst_r, dst_v, dst_s).reshape(2, 3, n, 16)


def _tc_mm_blocked(x, w):
    n, k = x.shape
    wo = w.shape[1]
    jc = wo // 32

    def body(x_ref, w_ref, o_ref):
        y = jnp.dot(x_ref[...], w_ref[...], preferred_element_type=jnp.float32)
        for t in range(jc):
            o_ref[t] = y[:, t * 32:(t + 1) * 32]

    return pl.pallas_call(
        body,
        grid=(n // _BN,),
        in_specs=[pl.BlockSpec((_BN, k), lambda i: (i, 0)),
                  pl.BlockSpec((k, wo), lambda i: (0, 0))],
        out_specs=pl.BlockSpec((jc, _BN, 32), lambda i: (0, i, 0)),
        out_shape=jax.ShapeDtypeStruct((jc, n, 32), jnp.float32),
    )(x, w)


def _tc_mm_from_blocked(hb, w):
    n = hb.shape[1]
    wo = w.shape[1]
    jc = wo // 32

    def body(h_ref, w_ref, o_ref):
        h = jnp.concatenate([h_ref[t] for t in range(4)], axis=1)
        y = jnp.dot(h, w_ref[...], preferred_element_type=jnp.float32)
        for t in range(jc):
            o_ref[t] = y[:, t * 32:(t + 1) * 32]

    return pl.pallas_call(
        body,
        grid=(n // _BN,),
        in_specs=[pl.BlockSpec((4, _BN, 32), lambda i: (0, i, 0)),
                  pl.BlockSpec((128, wo), lambda i: (0, 0))],
        out_specs=pl.BlockSpec((jc, _BN, 32), lambda i: (0, i, 0)),
        out_shape=jax.ShapeDtypeStruct((jc, n, 32), jnp.float32),
    )(hb, w)


def _inv_cnt(c0, c1):
    return 1.0 / jnp.maximum(c0[:, 0:1] + c1[:, 0:1], 1.0)


def _tc_combine1_dual(agg_a, ca0, ca1, agg_b, cb0, cb1, x, wa, wb, ba, bb):
    n = x.shape[0]

    def body(aa, a0, a1, ab, b0, b1, x_ref, wa_ref, wb_ref, ba_ref, bb_ref,
             o_ref):
        sa = jnp.concatenate([aa[t] for t in range(4)], axis=1)
        sb = jnp.concatenate([ab[t] for t in range(4)], axis=1)
        y = jnp.dot(x_ref[...], wa_ref[...] + wb_ref[...],
                    preferred_element_type=jnp.float32)
        res = (sa * _inv_cnt(a0, a1) + sb * _inv_cnt(b0, b1) + y
               + ba_ref[...] + bb_ref[...])
        res = jnp.maximum(res, 0.0)
        for t in range(4):
            o_ref[t] = res[:, t * 32:(t + 1) * 32]

    cspec = pl.BlockSpec((_BN, 16), lambda i: (i, 0))
    aspec = pl.BlockSpec((4, _BN, 32), lambda i: (0, i, 0))
    wspec = pl.BlockSpec((128, 128), lambda i: (0, 0))
    bspec = pl.BlockSpec((1, 128), lambda i: (0, 0))
    return pl.pallas_call(
        body,
        grid=(n // _BN,),
        in_specs=[aspec, cspec, cspec, aspec, cspec, cspec,
                  pl.BlockSpec((_BN, 128), lambda i: (i, 0)),
                  wspec, wspec, bspec, bspec],
        out_specs=pl.BlockSpec((4, _BN, 32), lambda i: (0, i, 0)),
        out_shape=jax.ShapeDtypeStruct((4, n, 32), jnp.float32),
    )(agg_a, ca0, ca1, agg_b, cb0, cb1, x, wa, wb, ba, bb)


def _tc_combine1_single(agg_a, ca0, ca1, x, wa, ba):
    n = x.shape[0]

    def body(aa, a0, a1, x_ref, wa_ref, ba_ref, o_ref):
        sa = jnp.concatenate([aa[t] for t in range(4)], axis=1)
        y = jnp.dot(x_ref[...], wa_ref[...],
                    preferred_element_type=jnp.float32)
        res = jnp.maximum(sa * _inv_cnt(a0, a1) + y + ba_ref[...], 0.0)
        for t in range(4):
            o_ref[t] = res[:, t * 32:(t + 1) * 32]

    cspec = pl.BlockSpec((_BN, 16), lambda i: (i, 0))
    return pl.pallas_call(
        body,
        grid=(n // _BN,),
        in_specs=[pl.BlockSpec((4, _BN, 32), lambda i: (0, i, 0)),
                  cspec, cspec,
                  pl.BlockSpec((_BN, 128), lambda i: (i, 0)),
                  pl.BlockSpec((128, 128), lambda i: (0, 0)),
                  pl.BlockSpec((1, 128), lambda i: (0, 0))],
        out_specs=pl.BlockSpec((4, _BN, 32), lambda i: (0, i, 0)),
        out_shape=jax.ShapeDtypeStruct((4, n, 32), jnp.float32),
    )(agg_a, ca0, ca1, x, wa, ba)


def _tc_combine2_dual(agg_a, ca0, ca1, agg_b, cb0, cb1, hb, wa, wb, ba, bb):
    n = hb.shape[1]

    def body(aa, a0, a1, ab, b0, b1, h_ref, wa_ref, wb_ref, ba_ref, bb_ref,
             o_ref):
        sa = jnp.concatenate([aa[0], aa[1]], axis=1)
        sb = jnp.concatenate([ab[0], ab[1]], axis=1)
        h = jnp.concatenate([h_ref[t] for t in range(4)], axis=1)
        y = jnp.dot(h, wa_ref[...] + wb_ref[...],
                    preferred_element_type=jnp.float32)
        o_ref[...] = (sa * _inv_cnt(a0, a1) + sb * _inv_cnt(b0, b1) + y
                      + ba_ref[...] + bb_ref[...])

    cspec = pl.BlockSpec((_BN, 16), lambda i: (i, 0))
    aspec = pl.BlockSpec((2, _BN, 32), lambda i: (0, i, 0))
    wspec = pl.BlockSpec((128, 64), lambda i: (0, 0))
    bspec = pl.BlockSpec((1, 64), lambda i: (0, 0))
    return pl.pallas_call(
        body,
        grid=(n // _BN,),
        in_specs=[aspec, cspec, cspec, aspec, cspec, cspec,
                  pl.BlockSpec((4, _BN, 32), lambda i: (0, i, 0)),
                  wspec, wspec, bspec, bspec],
        out_specs=pl.BlockSpec((_BN, 64), lambda i: (i, 0)),
        out_shape=jax.ShapeDtypeStruct((n, 64), jnp.float32),
    )(agg_a, ca0, ca1, agg_b, cb0, cb1, hb, wa, wb, ba, bb)


def _tc_combine2_single(agg_a, ca0, ca1, hb, wa, ba):
    n = hb.shape[1]

    def body(aa, a0, a1, h_ref, wa_ref, ba_ref, o_ref):
        sa = jnp.concatenate([aa[0], aa[1]], axis=1)
        h = jnp.concatenate([h_ref[t] for t in range(4)], axis=1)
        y = jnp.dot(h, wa_ref[...], preferred_element_type=jnp.float32)
        o_ref[...] = sa * _inv_cnt(a0, a1) + y + ba_ref[...]

    cspec = pl.BlockSpec((_BN, 16), lambda i: (i, 0))
    return pl.pallas_call(
        body,
        grid=(n // _BN,),
        in_specs=[pl.BlockSpec((2, _BN, 32), lambda i: (0, i, 0)),
                  cspec, cspec,
                  pl.BlockSpec((4, _BN, 32), lambda i: (0, i, 0)),
                  pl.BlockSpec((128, 64), lambda i: (0, 0)),
                  pl.BlockSpec((1, 64), lambda i: (0, 0))],
        out_specs=pl.BlockSpec((_BN, 64), lambda i: (i, 0)),
        out_shape=jax.ShapeDtypeStruct((n, 64), jnp.float32),
    )(agg_a, ca0, ca1, hb, wa, ba)


def kernel(x_user, x_product, ei_reviews, ei_rev_reviews, ei_similar,
           w1_rp_l, b1_rp, w1_rp_r, w1_pu_l, b1_pu, w1_pu_r,
           w1_pp_l, b1_pp, w1_pp_r,
           w2_rp_l, b2_rp, w2_rp_r, w2_pu_l, b2_pu, w2_pu_r,
           w2_pp_l, b2_pp, w2_pp_r):
    s_r, d_r = _prep(ei_reviews[0], ei_reviews[1])
    s_v, d_v = _prep(ei_rev_reviews[0], ei_rev_reviews[1])
    s_s, d_s = _prep(ei_similar[0], ei_similar[1])

    cparts = _sc_counts(d_r, d_v, d_s)
    cr0, cr1 = cparts[0, 0], cparts[1, 0]
    cv0, cv1 = cparts[0, 1], cparts[1, 1]
    cs0, cs1 = cparts[0, 2], cparts[1, 2]

    yu1 = _tc_mm_blocked(x_user, w1_rp_l)
    ypp1 = _tc_mm_blocked(x_product, w1_pp_l)
    ypu1 = _tc_mm_blocked(x_product, w1_pu_l)
    agg_r1 = _sc_agg(yu1.reshape(-1, 32), _adj(s_r, 4, _NU), d_r, 4, _NP)
    agg_s1 = _sc_agg(ypp1.reshape(-1, 32), _adj(s_s, 4, _NP), d_s, 4, _NP)
    agg_v1 = _sc_agg(ypu1.reshape(-1, 32), _adj(s_v, 4, _NP), d_v, 4, _NU)
    h_p = _tc_combine1_dual(agg_r1, cr0, cr1, agg_s1, cs0, cs1, x_product,
                            w1_rp_r, w1_pp_r,
                            b1_rp.reshape(1, -1), b1_pp.reshape(1, -1))
    h_u = _tc_combine1_single(agg_v1, cv0, cv1, x_user, w1_pu_r,
                              b1_pu.reshape(1, -1))

    y2r = _tc_mm_from_blocked(h_u, w2_rp_l)
    y2s = _tc_mm_from_blocked(h_p, w2_pp_l)
    y2v = _tc_mm_from_blocked(h_p, w2_pu_l)
    agg_r2 = _sc_agg(y2r.reshape(-1, 32), _adj(s_r, 2, _NU), d_r, 2, _NP)
    agg_s2 = _sc_agg(y2s.reshape(-1, 32), _adj(s_s, 2, _NP), d_s, 2, _NP)
    agg_v2 = _sc_agg(y2v.reshape(-1, 32), _adj(s_v, 2, _NP), d_v, 2, _NU)
    out_p = _tc_combine2_dual(agg_r2, cr0, cr1, agg_s2, cs0, cs1, h_p,
                              w2_rp_r, w2_pp_r,
                              b2_rp.reshape(1, -1), b2_pp.reshape(1, -1))
    out_u = _tc_combine2_single(agg_v2, cv0, cv1, h_u, w2_pu_r,
                                b2_pu.reshape(1, -1))
    return (out_u, out_p)


# speedup vs baseline: 2.7758x; 1.0383x over previous
import jax
import jax.numpy as jnp
from jax import lax
from jax.experimental import pallas as pl
from jax.experimental.pallas import tpu as pltpu
from jax.experimental.pallas import tpu_sc as plsc

_NU = 50000
_NP = 50000
_BN = 2000
_NBIG = 16
_NSUB = 16
_NACC = 50048
_TRASH = 50000
_WPS = 3128
_ZPS = _NACC // _NSUB


def _zero_acc(acc, zbuf, sid, zsem):
    def zacc(t, carry):
        pltpu.async_copy(zbuf, acc.at[pl.ds(sid * _ZPS + t * 128, 128)],
                         zsem)
        return carry
    lax.fori_loop(0, _ZPS // 128, zacc, 0)
    rem = _ZPS % 128
    pltpu.async_copy(zbuf.at[pl.ds(0, rem)],
                     acc.at[pl.ds(sid * _ZPS + _ZPS - rem, rem)], zsem)

    def zwait(t, carry):
        pltpu.make_async_copy(
            zbuf, acc.at[pl.ds(sid * _ZPS, 128)], zsem).wait()
        return carry
    lax.fori_loop(0, _ZPS // 128, zwait, 0)
    pltpu.make_async_copy(zbuf.at[pl.ds(0, rem)],
                          acc.at[pl.ds(sid * _ZPS, rem)], zsem).wait()


def _writeout(acc, out_ref, sid, out_base, n_dst):
    last = n_dst - _WPS

    @pl.when(sid < _NSUB - 1)
    def _():
        b = pl.multiple_of(sid * _WPS, 8)
        pltpu.sync_copy(acc.at[pl.ds(b, _WPS)],
                        out_ref.at[pl.ds(pl.multiple_of(out_base + b, 8),
                                         _WPS)])

    @pl.when(sid == _NSUB - 1)
    def _():
        pltpu.sync_copy(acc.at[pl.ds(last, _WPS)],
                        out_ref.at[pl.ds(pl.multiple_of(out_base + last, 8),
                                         _WPS)])


def _prep(src, dst):
    e = src.shape[0]
    nb = ((e + 32767) // 32768) * 256
    ep = nb * 128
    src = jnp.concatenate([src.astype(jnp.int32),
                           jnp.zeros((ep - e,), jnp.int32)])
    dst = jnp.concatenate([dst.astype(jnp.int32),
                           jnp.full((ep - e,), _TRASH, jnp.int32)])
    return src.reshape(nb, 128), dst.reshape(nb, 128)


def _adj(src2d, j_count, n_src):
    off = (jnp.arange(j_count, dtype=jnp.int32) * n_src)[:, None, None]
    return (src2d[None] + off).reshape(j_count * src2d.shape[0], 128)


_MESH = dict(core_axis_name="c", subcore_axis_name="s")


def _sc_agg(table, srcadj, dst2d, j_count, n_dst):
    nb = dst2d.shape[0]
    gps = nb // _NSUB
    nchunk = gps // _NBIG
    rounds = j_count // 2

    def body(table_ref, src_ref, dst_ref, out_ref, acc, sb0, db0, sb1, db1,
             r0, r1, r2, r3, g0, g1, g2, g3, s0, s1, s2, s3, esem):
        rows = (r0, r1, r2, r3)
        gsems = (g0, g1, g2, g3)
        ssems = (s0, s1, s2, s3)
        ebufs = ((sb0, db0), (sb1, db1))
        cid = lax.axis_index("c")
        sid = lax.axis_index("s")

        def load_chunk(j, cq, par):
            @pl.when(cq < nchunk)
            def _():
                base = sid * gps + cq * _NBIG
                pltpu.async_copy(src_ref.at[pl.ds(j * nb + base, _NBIG)],
                                 ebufs[par][0], esem)
                pltpu.async_copy(dst_ref.at[pl.ds(base, _NBIG)],
                                 ebufs[par][1], esem)

        def wait_chunk(par):
            pltpu.make_async_copy(src_ref.at[pl.ds(0, _NBIG)],
                                  ebufs[par][0], esem).wait()
            pltpu.make_async_copy(dst_ref.at[pl.ds(0, _NBIG)],
                                  ebufs[par][1], esem).wait()

        for r in range(rounds):
            j = cid + 2 * r

            def zrow(i, carry):
                r0[i, pl.ds(0, 16)] = jnp.zeros((16,), jnp.float32)
                r0[i, pl.ds(16, 16)] = jnp.zeros((16,), jnp.float32)
                return carry
            lax.fori_loop(0, 128, zrow, 0)
            _zero_acc(acc, r0, sid, g0)
            plsc.subcore_barrier()

            load_chunk(j, 0, 0)

            def chunk_pair(q, carry):
                for par in range(2):
                    cq = 2 * q + par
                    srcbig, dstbig = ebufs[par]
                    wait_chunk(par)

                    def sstep(ss, carry2):
                        for i in range(4):
                            k = 4 * ss + i

                            @pl.when((ss > 0) | (cq > 0))
                            def _():
                                pltpu.make_async_copy(
                                    rows[i], acc.at[dstbig.at[k]],
                                    ssems[i]).wait()
                            pltpu.async_copy(table_ref.at[srcbig.at[k]],
                                             rows[i], gsems[i])
                        for i in range(4):
                            k = 4 * ss + i
                            pltpu.make_async_copy(
                                table_ref.at[srcbig.at[k]], rows[i],
                                gsems[i]).wait()
                            pltpu.async_copy(rows[i], acc.at[dstbig.at[k]],
                                             ssems[i], add=True)
                        return carry2
                    sstep(0, 0)
                    load_chunk(j, cq + 1, 1 - par)
                    lax.fori_loop(1, _NBIG // 4, sstep, 0)
                return carry
            lax.fori_loop(0, nchunk // 2, chunk_pair, 0)
            for i in range(4):
                pltpu.make_async_copy(rows[i], acc.at[db1.at[i]],
                                      ssems[i]).wait()
            plsc.subcore_barrier()

            _writeout(acc, out_ref, sid, j * n_dst, n_dst)
            plsc.subcore_barrier()

    f = pl.kernel(
        body,
        out_type=jax.ShapeDtypeStruct((j_count * n_dst, 32), jnp.float32),
        mesh=plsc.VectorSubcoreMesh(**_MESH),
        compiler_params=pltpu.CompilerParams(use_tc_tiling_on_sc=False),
        scratch_types=(
            [pltpu.VMEM_SHARED((_NACC, 32), jnp.float32)]
            + [pltpu.VMEM((_NBIG, 128), jnp.int32) for _ in range(4)]
            + [pltpu.VMEM((128, 32), jnp.float32) for _ in range(4)]
            + [pltpu.SemaphoreType.DMA] * 9
        ),
    )
    return f(table, srcadj, dst2d).reshape(j_count, n_dst, 32)


def _sc_counts(dst_r, dst_v, dst_s):
    n = _NP
    cz = 3200
    ncnt = _NSUB * cz

    def body(dr_ref, dv_ref, ds_ref, out_ref, cnt, stage, dstbig, tmp, red):
        cid = lax.axis_index("c")
        sid = lax.axis_index("s")
        wid = cid * _NSUB + sid
        ones = jnp.ones((16,), jnp.float32)

        for t, dref in ((0, dr_ref), (1, dv_ref), (2, ds_ref)):
            nb = dref.shape[0]
            gpw = nb // (2 * _NSUB)

            def zc(i, carry):
                cnt[pl.ds(i * 16, 16)] = jnp.zeros((16,), jnp.float32)
                return carry
            lax.fori_loop(0, ncnt // 16, zc, 0)

            def chunk(cq, carry):
                gb = wid * gpw + cq * _NBIG
                pltpu.sync_copy(dref.at[pl.ds(gb, _NBIG)], dstbig)

                def grp(g, carry2):
                    for h in range(8):
                        iv = dstbig[g, pl.ds(16 * h, 16)]
                        plsc.addupdate_scatter(cnt, [iv], ones)
                    return carry2
                lax.fori_loop(0, _NBIG, grp, 0)
                return carry
            lax.fori_loop(0, gpw // _NBIG, chunk, 0)

            pltpu.sync_copy(cnt, stage.at[sid])
            plsc.subcore_barrier()

            def zr(i, carry):
                red[pl.ds(i * 16, 16)] = jnp.zeros((16,), jnp.float32)
                return carry
            lax.fori_loop(0, cz // 16, zr, 0)
            for tt in range(_NSUB):
                pltpu.sync_copy(stage.at[tt, pl.ds(sid * cz, cz)], tmp)

                def radd(i, carry):
                    red[pl.ds(i * 16, 16)] = (red[pl.ds(i * 16, 16)]
                                              + tmp[pl.ds(i * 16, 16)])
                    return carry
                lax.fori_loop(0, cz // 16, radd, 0)

            obase = cid * 3 * n + t * n

            @pl.when(sid < _NSUB - 1)
            def _():
                pltpu.sync_copy(
                    red, out_ref.at[pl.ds(obase + sid * cz, cz)])

            @pl.when(sid == _NSUB - 1)
            def _():
                pltpu.sync_copy(
                    red.at[pl.ds(0, n - 15 * cz)],
                    out_ref.at[pl.ds(obase + 15 * cz, n - 15 * cz)])
            plsc.subcore_barrier()

    f = pl.kernel(
        body,
        out_type=jax.ShapeDtypeStruct((2 * 3 * n,), jnp.float32),
        mesh=plsc.VectorSubcoreMesh(**_MESH),
        compiler_params=pltpu.CompilerParams(use_tc_tiling_on_sc=False,
                                             needs_layout_passes=False),
        scratch_types=[
            pltpu.VMEM((_NSUB * 3200,), jnp.float32),
            pltpu.VMEM_SHARED((_NSUB, _NSUB * 3200), jnp.float32),
            pltpu.VMEM((_NBIG, 128), jnp.int32),
            pltpu.VMEM((3200,), jnp.float32),
            pltpu.VMEM((3200,), jnp.float32),
        ],
    )
    parts = f(dst_r, dst_v, dst_s).reshape(2, 3, n)
    return jnp.broadcast_to(parts[:, :, :, None], (2, 3, n, 16))


def _tc_mm_blocked(x, w):
    n, k = x.shape
    wo = w.shape[1]
    jc = wo // 32

    def body(x_ref, w_ref, o_ref):
        y = jnp.dot(x_ref[...], w_ref[...], preferred_element_type=jnp.float32)
        for t in range(jc):
            o_ref[t] = y[:, t * 32:(t + 1) * 32]

    return pl.pallas_call(
        body,
        grid=(n // _BN,),
        in_specs=[pl.BlockSpec((_BN, k), lambda i: (i, 0)),
                  pl.BlockSpec((k, wo), lambda i: (0, 0))],
        out_specs=pl.BlockSpec((jc, _BN, 32), lambda i: (0, i, 0)),
        out_shape=jax.ShapeDtypeStruct((jc, n, 32), jnp.float32),
    )(x, w)


def _tc_mm_from_blocked(hb, w):
    n = hb.shape[1]
    wo = w.shape[1]
    jc = wo // 32

    def body(h_ref, w_ref, o_ref):
        h = jnp.concatenate([h_ref[t] for t in range(4)], axis=1)
        y = jnp.dot(h, w_ref[...], preferred_element_type=jnp.float32)
        for t in range(jc):
            o_ref[t] = y[:, t * 32:(t + 1) * 32]

    return pl.pallas_call(
        body,
        grid=(n // _BN,),
        in_specs=[pl.BlockSpec((4, _BN, 32), lambda i: (0, i, 0)),
                  pl.BlockSpec((128, wo), lambda i: (0, 0))],
        out_specs=pl.BlockSpec((jc, _BN, 32), lambda i: (0, i, 0)),
        out_shape=jax.ShapeDtypeStruct((jc, n, 32), jnp.float32),
    )(hb, w)


def _inv_cnt(c0, c1):
    return 1.0 / jnp.maximum(c0[:, 0:1] + c1[:, 0:1], 1.0)


def _tc_combine1_dual(agg_a, ca0, ca1, agg_b, cb0, cb1, x, wa, wb, ba, bb):
    n = x.shape[0]

    def body(aa, a0, a1, ab, b0, b1, x_ref, wa_ref, wb_ref, ba_ref, bb_ref,
             o_ref):
        sa = jnp.concatenate([aa[t] for t in range(4)], axis=1)
        sb = jnp.concatenate([ab[t] for t in range(4)], axis=1)
        y = jnp.dot(x_ref[...], wa_ref[...] + wb_ref[...],
                    preferred_element_type=jnp.float32)
        res = (sa * _inv_cnt(a0, a1) + sb * _inv_cnt(b0, b1) + y
               + ba_ref[...] + bb_ref[...])
        res = jnp.maximum(res, 0.0)
        for t in range(4):
            o_ref[t] = res[:, t * 32:(t + 1) * 32]

    cspec = pl.BlockSpec((_BN, 16), lambda i: (i, 0))
    aspec = pl.BlockSpec((4, _BN, 32), lambda i: (0, i, 0))
    wspec = pl.BlockSpec((128, 128), lambda i: (0, 0))
    bspec = pl.BlockSpec((1, 128), lambda i: (0, 0))
    return pl.pallas_call(
        body,
        grid=(n // _BN,),
        in_specs=[aspec, cspec, cspec, aspec, cspec, cspec,
                  pl.BlockSpec((_BN, 128), lambda i: (i, 0)),
                  wspec, wspec, bspec, bspec],
        out_specs=pl.BlockSpec((4, _BN, 32), lambda i: (0, i, 0)),
        out_shape=jax.ShapeDtypeStruct((4, n, 32), jnp.float32),
    )(agg_a, ca0, ca1, agg_b, cb0, cb1, x, wa, wb, ba, bb)


def _tc_combine1_single(agg_a, ca0, ca1, x, wa, ba):
    n = x.shape[0]

    def body(aa, a0, a1, x_ref, wa_ref, ba_ref, o_ref):
        sa = jnp.concatenate([aa[t] for t in range(4)], axis=1)
        y = jnp.dot(x_ref[...], wa_ref[...],
                    preferred_element_type=jnp.float32)
        res = jnp.maximum(sa * _inv_cnt(a0, a1) + y + ba_ref[...], 0.0)
        for t in range(4):
            o_ref[t] = res[:, t * 32:(t + 1) * 32]

    cspec = pl.BlockSpec((_BN, 16), lambda i: (i, 0))
    return pl.pallas_call(
        body,
        grid=(n // _BN,),
        in_specs=[pl.BlockSpec((4, _BN, 32), lambda i: (0, i, 0)),
                  cspec, cspec,
                  pl.BlockSpec((_BN, 128), lambda i: (i, 0)),
                  pl.BlockSpec((128, 128), lambda i: (0, 0)),
                  pl.BlockSpec((1, 128), lambda i: (0, 0))],
        out_specs=pl.BlockSpec((4, _BN, 32), lambda i: (0, i, 0)),
        out_shape=jax.ShapeDtypeStruct((4, n, 32), jnp.float32),
    )(agg_a, ca0, ca1, x, wa, ba)


def _tc_combine2_dual(agg_a, ca0, ca1, agg_b, cb0, cb1, hb, wa, wb, ba, bb):
    n = hb.shape[1]

    def body(aa, a0, a1, ab, b0, b1, h_ref, wa_ref, wb_ref, ba_ref, bb_ref,
             o_ref):
        sa = jnp.concatenate([aa[0], aa[1]], axis=1)
        sb = jnp.concatenate([ab[0], ab[1]], axis=1)
        h = jnp.concatenate([h_ref[t] for t in range(4)], axis=1)
        y = jnp.dot(h, wa_ref[...] + wb_ref[...],
                    preferred_element_type=jnp.float32)
        o_ref[...] = (sa * _inv_cnt(a0, a1) + sb * _inv_cnt(b0, b1) + y
                      + ba_ref[...] + bb_ref[...])

    cspec = pl.BlockSpec((_BN, 16), lambda i: (i, 0))
    aspec = pl.BlockSpec((2, _BN, 32), lambda i: (0, i, 0))
    wspec = pl.BlockSpec((128, 64), lambda i: (0, 0))
    bspec = pl.BlockSpec((1, 64), lambda i: (0, 0))
    return pl.pallas_call(
        body,
        grid=(n // _BN,),
        in_specs=[aspec, cspec, cspec, aspec, cspec, cspec,
                  pl.BlockSpec((4, _BN, 32), lambda i: (0, i, 0)),
                  wspec, wspec, bspec, bspec],
        out_specs=pl.BlockSpec((_BN, 64), lambda i: (i, 0)),
        out_shape=jax.ShapeDtypeStruct((n, 64), jnp.float32),
    )(agg_a, ca0, ca1, agg_b, cb0, cb1, hb, wa, wb, ba, bb)


def _tc_combine2_single(agg_a, ca0, ca1, hb, wa, ba):
    n = hb.shape[1]

    def body(aa, a0, a1, h_ref, wa_ref, ba_ref, o_ref):
        sa = jnp.concatenate([aa[0], aa[1]], axis=1)
        h = jnp.concatenate([h_ref[t] for t in range(4)], axis=1)
        y = jnp.dot(h, wa_ref[...], preferred_element_type=jnp.float32)
        o_ref[...] = sa * _inv_cnt(a0, a1) + y + ba_ref[...]

    cspec = pl.BlockSpec((_BN, 16), lambda i: (i, 0))
    return pl.pallas_call(
        body,
        grid=(n // _BN,),
        in_specs=[pl.BlockSpec((2, _BN, 32), lambda i: (0, i, 0)),
                  cspec, cspec,
                  pl.BlockSpec((4, _BN, 32), lambda i: (0, i, 0)),
                  pl.BlockSpec((128, 64), lambda i: (0, 0)),
                  pl.BlockSpec((1, 64), lambda i: (0, 0))],
        out_specs=pl.BlockSpec((_BN, 64), lambda i: (i, 0)),
        out_shape=jax.ShapeDtypeStruct((n, 64), jnp.float32),
    )(agg_a, ca0, ca1, hb, wa, ba)


def kernel(x_user, x_product, ei_reviews, ei_rev_reviews, ei_similar,
           w1_rp_l, b1_rp, w1_rp_r, w1_pu_l, b1_pu, w1_pu_r,
           w1_pp_l, b1_pp, w1_pp_r,
           w2_rp_l, b2_rp, w2_rp_r, w2_pu_l, b2_pu, w2_pu_r,
           w2_pp_l, b2_pp, w2_pp_r):
    s_r, d_r = _prep(ei_reviews[0], ei_reviews[1])
    s_v, d_v = _prep(ei_rev_reviews[0], ei_rev_reviews[1])
    s_s, d_s = _prep(ei_similar[0], ei_similar[1])

    cparts = _sc_counts(d_r, d_v, d_s)
    cr0, cr1 = cparts[0, 0], cparts[1, 0]
    cv0, cv1 = cparts[0, 1], cparts[1, 1]
    cs0, cs1 = cparts[0, 2], cparts[1, 2]

    yu1 = _tc_mm_blocked(x_user, w1_rp_l)
    ypp1 = _tc_mm_blocked(x_product, w1_pp_l)
    ypu1 = _tc_mm_blocked(x_product, w1_pu_l)
    agg_r1 = _sc_agg(yu1.reshape(-1, 32), _adj(s_r, 4, _NU), d_r, 4, _NP)
    agg_s1 = _sc_agg(ypp1.reshape(-1, 32), _adj(s_s, 4, _NP), d_s, 4, _NP)
    agg_v1 = _sc_agg(ypu1.reshape(-1, 32), _adj(s_v, 4, _NP), d_v, 4, _NU)
    h_p = _tc_combine1_dual(agg_r1, cr0, cr1, agg_s1, cs0, cs1, x_product,
                            w1_rp_r, w1_pp_r,
                            b1_rp.reshape(1, -1), b1_pp.reshape(1, -1))
    h_u = _tc_combine1_single(agg_v1, cv0, cv1, x_user, w1_pu_r,
                              b1_pu.reshape(1, -1))

    y2r = _tc_mm_from_blocked(h_u, w2_rp_l)
    y2s = _tc_mm_from_blocked(h_p, w2_pp_l)
    y2v = _tc_mm_from_blocked(h_p, w2_pu_l)
    agg_r2 = _sc_agg(y2r.reshape(-1, 32), _adj(s_r, 2, _NU), d_r, 2, _NP)
    agg_s2 = _sc_agg(y2s.reshape(-1, 32), _adj(s_s, 2, _NP), d_s, 2, _NP)
    agg_v2 = _sc_agg(y2v.reshape(-1, 32), _adj(s_v, 2, _NP), d_v, 2, _NU)
    out_p = _tc_combine2_dual(agg_r2, cr0, cr1, agg_s2, cs0, cs1, h_p,
                              w2_rp_r, w2_pp_r,
                              b2_rp.reshape(1, -1), b2_pp.reshape(1, -1))
    out_u = _tc_combine2_single(agg_v2, cv0, cv1, h_u, w2_pu_r,
                                b2_pu.reshape(1, -1))
    return (out_u, out_p)


# speedup vs baseline: 2.8350x; 1.0213x over previous
import jax
import jax.numpy as jnp
from jax import lax
from jax.experimental import pallas as pl
from jax.experimental.pallas import tpu as pltpu
from jax.experimental.pallas import tpu_sc as plsc

_NU = 50000
_NP = 50000
_BN = 2000
_NBIG = 16
_NSUB = 16
_NACC = 50048
_TRASH = 50000
_WPS = 3128
_ZPS = _NACC // _NSUB


def _zero_acc(acc, zbuf, sid, zsem):
    def zacc(t, carry):
        pltpu.async_copy(zbuf, acc.at[pl.ds(sid * _ZPS + t * 128, 128)],
                         zsem)
        return carry
    lax.fori_loop(0, _ZPS // 128, zacc, 0)
    rem = _ZPS % 128
    pltpu.async_copy(zbuf.at[pl.ds(0, rem)],
                     acc.at[pl.ds(sid * _ZPS + _ZPS - rem, rem)], zsem)

    def zwait(t, carry):
        pltpu.make_async_copy(
            zbuf, acc.at[pl.ds(sid * _ZPS, 128)], zsem).wait()
        return carry
    lax.fori_loop(0, _ZPS // 128, zwait, 0)
    pltpu.make_async_copy(zbuf.at[pl.ds(0, rem)],
                          acc.at[pl.ds(sid * _ZPS, rem)], zsem).wait()


def _writeout(acc, out_ref, sid, out_base, n_dst):
    last = n_dst - _WPS

    @pl.when(sid < _NSUB - 1)
    def _():
        b = pl.multiple_of(sid * _WPS, 8)
        pltpu.sync_copy(acc.at[pl.ds(b, _WPS)],
                        out_ref.at[pl.ds(pl.multiple_of(out_base + b, 8),
                                         _WPS)])

    @pl.when(sid == _NSUB - 1)
    def _():
        pltpu.sync_copy(acc.at[pl.ds(last, _WPS)],
                        out_ref.at[pl.ds(pl.multiple_of(out_base + last, 8),
                                         _WPS)])


def _prep(src, dst):
    e = src.shape[0]
    nb = ((e + 32767) // 32768) * 256
    ep = nb * 128
    src = jnp.concatenate([src.astype(jnp.int32),
                           jnp.zeros((ep - e,), jnp.int32)])
    dst = jnp.concatenate([dst.astype(jnp.int32),
                           jnp.full((ep - e,), _TRASH, jnp.int32)])
    return src.reshape(nb, 128), dst.reshape(nb, 128)


def _adj(src2d, j_count, n_src):
    off = (jnp.arange(j_count, dtype=jnp.int32) * n_src)[:, None, None]
    return (src2d[None] + off).reshape(j_count * src2d.shape[0], 128)


_MESH = dict(core_axis_name="c", subcore_axis_name="s")


def _sc_agg(table, srcadj, dst2d, j_count, n_dst):
    nb = dst2d.shape[0]
    gps = nb // _NSUB
    nchunk = gps // _NBIG
    rounds = j_count // 2

    def body(table_ref, src_ref, dst_ref, out_ref, acc, sb0, db0, sb1, db1,
             r0, r1, r2, r3, g0, g1, g2, g3, s0, s1, s2, s3, esem):
        rows = (r0, r1, r2, r3)
        gsems = (g0, g1, g2, g3)
        ssems = (s0, s1, s2, s3)
        ebufs = ((sb0, db0), (sb1, db1))
        cid = lax.axis_index("c")
        sid = lax.axis_index("s")

        def load_chunk(j, cq, par):
            @pl.when(cq < nchunk)
            def _():
                base = sid * gps + cq * _NBIG
                pltpu.async_copy(src_ref.at[pl.ds(j * nb + base, _NBIG)],
                                 ebufs[par][0], esem)
                pltpu.async_copy(dst_ref.at[pl.ds(base, _NBIG)],
                                 ebufs[par][1], esem)

        def wait_chunk(par):
            pltpu.make_async_copy(src_ref.at[pl.ds(0, _NBIG)],
                                  ebufs[par][0], esem).wait()
            pltpu.make_async_copy(dst_ref.at[pl.ds(0, _NBIG)],
                                  ebufs[par][1], esem).wait()

        for r in range(rounds):
            j = cid + 2 * r

            def zrow(i, carry):
                r0[i, pl.ds(0, 16)] = jnp.zeros((16,), jnp.float32)
                r0[i, pl.ds(16, 16)] = jnp.zeros((16,), jnp.float32)
                return carry
            lax.fori_loop(0, 128, zrow, 0)
            _zero_acc(acc, r0, sid, g0)
            plsc.subcore_barrier()

            load_chunk(j, 0, 0)

            def chunk_pair(q, carry):
                for par in range(2):
                    cq = 2 * q + par
                    srcbig, dstbig = ebufs[par]
                    wait_chunk(par)

                    def sstep(ss, carry2):
                        for i in range(4):
                            k = 4 * ss + i

                            @pl.when((ss > 0) | (cq > 0))
                            def _():
                                pltpu.make_async_copy(
                                    rows[i], acc.at[dstbig.at[k]],
                                    ssems[i]).wait()
                            pltpu.async_copy(table_ref.at[srcbig.at[k]],
                                             rows[i], gsems[i])
                        for i in range(4):
                            k = 4 * ss + i
                            pltpu.make_async_copy(
                                table_ref.at[srcbig.at[k]], rows[i],
                                gsems[i]).wait()
                            pltpu.async_copy(rows[i], acc.at[dstbig.at[k]],
                                             ssems[i], add=True)
                        return carry2
                    sstep(0, 0)
                    load_chunk(j, cq + 1, 1 - par)
                    lax.fori_loop(1, _NBIG // 4, sstep, 0)
                return carry
            lax.fori_loop(0, nchunk // 2, chunk_pair, 0)
            for i in range(4):
                pltpu.make_async_copy(rows[i], acc.at[db1.at[i]],
                                      ssems[i]).wait()
            plsc.subcore_barrier()

            _writeout(acc, out_ref, sid, j * n_dst, n_dst)
            plsc.subcore_barrier()

    f = pl.kernel(
        body,
        out_type=jax.ShapeDtypeStruct((j_count * n_dst, 32), jnp.float32),
        mesh=plsc.VectorSubcoreMesh(**_MESH),
        compiler_params=pltpu.CompilerParams(use_tc_tiling_on_sc=False),
        scratch_types=(
            [pltpu.VMEM_SHARED((_NACC, 32), jnp.float32)]
            + [pltpu.VMEM((_NBIG, 128), jnp.int32) for _ in range(4)]
            + [pltpu.VMEM((128, 32), jnp.float32) for _ in range(4)]
            + [pltpu.SemaphoreType.DMA] * 9
        ),
    )
    return f(table, srcadj, dst2d).reshape(j_count, n_dst, 32)


def _sc_counts(dst_r, dst_v, dst_s):
    n = _NP
    cz = 3200
    ncnt = _NSUB * cz

    def body(dr_ref, dv_ref, ds_ref, out_ref, cnt, stage, dstbig, tmp, red):
        cid = lax.axis_index("c")
        sid = lax.axis_index("s")
        wid = cid * _NSUB + sid
        ones = jnp.ones((16,), jnp.float32)

        for t, dref in ((0, dr_ref), (1, dv_ref), (2, ds_ref)):
            nb = dref.shape[0]
            gpw = nb // (2 * _NSUB)

            def zc(i, carry):
                cnt[pl.ds(i * 16, 16)] = jnp.zeros((16,), jnp.float32)
                return carry
            lax.fori_loop(0, ncnt // 16, zc, 0)

            def chunk(cq, carry):
                gb = wid * gpw + cq * _NBIG
                pltpu.sync_copy(dref.at[pl.ds(gb, _NBIG)], dstbig)

                def grp(g, carry2):
                    for h in range(8):
                        iv = dstbig[g, pl.ds(16 * h, 16)]
                        plsc.addupdate_scatter(cnt, [iv], ones)
                    return carry2
                lax.fori_loop(0, _NBIG, grp, 0)
                return carry
            lax.fori_loop(0, gpw // _NBIG, chunk, 0)

            pltpu.sync_copy(cnt, stage.at[sid])
            plsc.subcore_barrier()

            def zr(i, carry):
                red[pl.ds(i * 16, 16)] = jnp.zeros((16,), jnp.float32)
                return carry
            lax.fori_loop(0, cz // 16, zr, 0)
            for tt in range(_NSUB):
                pltpu.sync_copy(stage.at[tt, pl.ds(sid * cz, cz)], tmp)

                def radd(i, carry):
                    red[pl.ds(i * 16, 16)] = (red[pl.ds(i * 16, 16)]
                                              + tmp[pl.ds(i * 16, 16)])
                    return carry
                lax.fori_loop(0, cz // 16, radd, 0)

            obase = cid * 3 * n + t * n

            @pl.when(sid < _NSUB - 1)
            def _():
                pltpu.sync_copy(
                    red, out_ref.at[pl.ds(obase + sid * cz, cz)])

            @pl.when(sid == _NSUB - 1)
            def _():
                pltpu.sync_copy(
                    red.at[pl.ds(0, n - 15 * cz)],
                    out_ref.at[pl.ds(obase + 15 * cz, n - 15 * cz)])
            plsc.subcore_barrier()

    f = pl.kernel(
        body,
        out_type=jax.ShapeDtypeStruct((2 * 3 * n,), jnp.float32),
        mesh=plsc.VectorSubcoreMesh(**_MESH),
        compiler_params=pltpu.CompilerParams(use_tc_tiling_on_sc=False,
                                             needs_layout_passes=False),
        scratch_types=[
            pltpu.VMEM((_NSUB * 3200,), jnp.float32),
            pltpu.VMEM_SHARED((_NSUB, _NSUB * 3200), jnp.float32),
            pltpu.VMEM((_NBIG, 128), jnp.int32),
            pltpu.VMEM((3200,), jnp.float32),
            pltpu.VMEM((3200,), jnp.float32),
        ],
    )
    parts = f(dst_r, dst_v, dst_s).reshape(2, 3, n)
    return jnp.broadcast_to(parts[:, :, :, None], (2, 3, n, 16))


def _tc_mm_blocked(x, w):
    n, k = x.shape
    wo = w.shape[1]
    jc = wo // 32

    def body(x_ref, w_ref, o_ref):
        y = jnp.dot(x_ref[...], w_ref[...], preferred_element_type=jnp.float32)
        for t in range(jc):
            o_ref[t] = y[:, t * 32:(t + 1) * 32]

    return pl.pallas_call(
        body,
        grid=(n // _BN,),
        in_specs=[pl.BlockSpec((_BN, k), lambda i: (i, 0)),
                  pl.BlockSpec((k, wo), lambda i: (0, 0))],
        out_specs=pl.BlockSpec((jc, _BN, 32), lambda i: (0, i, 0)),
        out_shape=jax.ShapeDtypeStruct((jc, n, 32), jnp.float32),
    )(x, w)


def _inv_cnt(c0, c1):
    return 1.0 / jnp.maximum(c0[:, 0:1] + c1[:, 0:1], 1.0)


def _tc_combine1_dual(agg_a, ca0, ca1, agg_b, cb0, cb1, x, wa, wb, ba, bb,
                      w2a, w2b):
    n = x.shape[0]

    def body(aa, a0, a1, ab, b0, b1, x_ref, wa_ref, wb_ref, ba_ref, bb_ref,
             w2a_ref, w2b_ref, o_ref, o2a_ref, o2b_ref):
        sa = jnp.concatenate([aa[t] for t in range(4)], axis=1)
        sb = jnp.concatenate([ab[t] for t in range(4)], axis=1)
        y = jnp.dot(x_ref[...], wa_ref[...] + wb_ref[...],
                    preferred_element_type=jnp.float32)
        res = (sa * _inv_cnt(a0, a1) + sb * _inv_cnt(b0, b1) + y
               + ba_ref[...] + bb_ref[...])
        res = jnp.maximum(res, 0.0)
        for t in range(4):
            o_ref[t] = res[:, t * 32:(t + 1) * 32]
        y2a = jnp.dot(res, w2a_ref[...], preferred_element_type=jnp.float32)
        y2b = jnp.dot(res, w2b_ref[...], preferred_element_type=jnp.float32)
        for t in range(2):
            o2a_ref[t] = y2a[:, t * 32:(t + 1) * 32]
            o2b_ref[t] = y2b[:, t * 32:(t + 1) * 32]

    cspec = pl.BlockSpec((_BN, 16), lambda i: (i, 0))
    aspec = pl.BlockSpec((4, _BN, 32), lambda i: (0, i, 0))
    wspec = pl.BlockSpec((128, 128), lambda i: (0, 0))
    bspec = pl.BlockSpec((1, 128), lambda i: (0, 0))
    w2spec = pl.BlockSpec((128, 64), lambda i: (0, 0))
    o2spec = pl.BlockSpec((2, _BN, 32), lambda i: (0, i, 0))
    return pl.pallas_call(
        body,
        grid=(n // _BN,),
        in_specs=[aspec, cspec, cspec, aspec, cspec, cspec,
                  pl.BlockSpec((_BN, 128), lambda i: (i, 0)),
                  wspec, wspec, bspec, bspec, w2spec, w2spec],
        out_specs=[pl.BlockSpec((4, _BN, 32), lambda i: (0, i, 0)),
                   o2spec, o2spec],
        out_shape=[jax.ShapeDtypeStruct((4, n, 32), jnp.float32),
                   jax.ShapeDtypeStruct((2, n, 32), jnp.float32),
                   jax.ShapeDtypeStruct((2, n, 32), jnp.float32)],
    )(agg_a, ca0, ca1, agg_b, cb0, cb1, x, wa, wb, ba, bb, w2a, w2b)


def _tc_combine1_single(agg_a, ca0, ca1, x, wa, ba, w2a):
    n = x.shape[0]

    def body(aa, a0, a1, x_ref, wa_ref, ba_ref, w2a_ref, o_ref, o2a_ref):
        sa = jnp.concatenate([aa[t] for t in range(4)], axis=1)
        y = jnp.dot(x_ref[...], wa_ref[...],
                    preferred_element_type=jnp.float32)
        res = jnp.maximum(sa * _inv_cnt(a0, a1) + y + ba_ref[...], 0.0)
        for t in range(4):
            o_ref[t] = res[:, t * 32:(t + 1) * 32]
        y2a = jnp.dot(res, w2a_ref[...], preferred_element_type=jnp.float32)
        for t in range(2):
            o2a_ref[t] = y2a[:, t * 32:(t + 1) * 32]

    cspec = pl.BlockSpec((_BN, 16), lambda i: (i, 0))
    return pl.pallas_call(
        body,
        grid=(n // _BN,),
        in_specs=[pl.BlockSpec((4, _BN, 32), lambda i: (0, i, 0)),
                  cspec, cspec,
                  pl.BlockSpec((_BN, 128), lambda i: (i, 0)),
                  pl.BlockSpec((128, 128), lambda i: (0, 0)),
                  pl.BlockSpec((1, 128), lambda i: (0, 0)),
                  pl.BlockSpec((128, 64), lambda i: (0, 0))],
        out_specs=[pl.BlockSpec((4, _BN, 32), lambda i: (0, i, 0)),
                   pl.BlockSpec((2, _BN, 32), lambda i: (0, i, 0))],
        out_shape=[jax.ShapeDtypeStruct((4, n, 32), jnp.float32),
                   jax.ShapeDtypeStruct((2, n, 32), jnp.float32)],
    )(agg_a, ca0, ca1, x, wa, ba, w2a)


def _tc_combine2_dual(agg_a, ca0, ca1, agg_b, cb0, cb1, hb, wa, wb, ba, bb):
    n = hb.shape[1]

    def body(aa, a0, a1, ab, b0, b1, h_ref, wa_ref, wb_ref, ba_ref, bb_ref,
             o_ref):
        sa = jnp.concatenate([aa[0], aa[1]], axis=1)
        sb = jnp.concatenate([ab[0], ab[1]], axis=1)
        h = jnp.concatenate([h_ref[t] for t in range(4)], axis=1)
        y = jnp.dot(h, wa_ref[...] + wb_ref[...],
                    preferred_element_type=jnp.float32)
        o_ref[...] = (sa * _inv_cnt(a0, a1) + sb * _inv_cnt(b0, b1) + y
                      + ba_ref[...] + bb_ref[...])

    cspec = pl.BlockSpec((_BN, 16), lambda i: (i, 0))
    aspec = pl.BlockSpec((2, _BN, 32), lambda i: (0, i, 0))
    wspec = pl.BlockSpec((128, 64), lambda i: (0, 0))
    bspec = pl.BlockSpec((1, 64), lambda i: (0, 0))
    return pl.pallas_call(
        body,
        grid=(n // _BN,),
        in_specs=[aspec, cspec, cspec, aspec, cspec, cspec,
                  pl.BlockSpec((4, _BN, 32), lambda i: (0, i, 0)),
                  wspec, wspec, bspec, bspec],
        out_specs=pl.BlockSpec((_BN, 64), lambda i: (i, 0)),
        out_shape=jax.ShapeDtypeStruct((n, 64), jnp.float32),
    )(agg_a, ca0, ca1, agg_b, cb0, cb1, hb, wa, wb, ba, bb)


def _tc_combine2_single(agg_a, ca0, ca1, hb, wa, ba):
    n = hb.shape[1]

    def body(aa, a0, a1, h_ref, wa_ref, ba_ref, o_ref):
        sa = jnp.concatenate([aa[0], aa[1]], axis=1)
        h = jnp.concatenate([h_ref[t] for t in range(4)], axis=1)
        y = jnp.dot(h, wa_ref[...], preferred_element_type=jnp.float32)
        o_ref[...] = sa * _inv_cnt(a0, a1) + y + ba_ref[...]

    cspec = pl.BlockSpec((_BN, 16), lambda i: (i, 0))
    return pl.pallas_call(
        body,
        grid=(n // _BN,),
        in_specs=[pl.BlockSpec((2, _BN, 32), lambda i: (0, i, 0)),
                  cspec, cspec,
                  pl.BlockSpec((4, _BN, 32), lambda i: (0, i, 0)),
                  pl.BlockSpec((128, 64), lambda i: (0, 0)),
                  pl.BlockSpec((1, 64), lambda i: (0, 0))],
        out_specs=pl.BlockSpec((_BN, 64), lambda i: (i, 0)),
        out_shape=jax.ShapeDtypeStruct((n, 64), jnp.float32),
    )(agg_a, ca0, ca1, hb, wa, ba)


def kernel(x_user, x_product, ei_reviews, ei_rev_reviews, ei_similar,
           w1_rp_l, b1_rp, w1_rp_r, w1_pu_l, b1_pu, w1_pu_r,
           w1_pp_l, b1_pp, w1_pp_r,
           w2_rp_l, b2_rp, w2_rp_r, w2_pu_l, b2_pu, w2_pu_r,
           w2_pp_l, b2_pp, w2_pp_r):
    s_r, d_r = _prep(ei_reviews[0], ei_reviews[1])
    s_v, d_v = _prep(ei_rev_reviews[0], ei_rev_reviews[1])
    s_s, d_s = _prep(ei_similar[0], ei_similar[1])

    cparts = _sc_counts(d_r, d_v, d_s)
    cr0, cr1 = cparts[0, 0], cparts[1, 0]
    cv0, cv1 = cparts[0, 1], cparts[1, 1]
    cs0, cs1 = cparts[0, 2], cparts[1, 2]

    yu1 = _tc_mm_blocked(x_user, w1_rp_l)
    ypp1 = _tc_mm_blocked(x_product, w1_pp_l)
    ypu1 = _tc_mm_blocked(x_product, w1_pu_l)
    agg_r1 = _sc_agg(yu1.reshape(-1, 32), _adj(s_r, 4, _NU), d_r, 4, _NP)
    agg_s1 = _sc_agg(ypp1.reshape(-1, 32), _adj(s_s, 4, _NP), d_s, 4, _NP)
    agg_v1 = _sc_agg(ypu1.reshape(-1, 32), _adj(s_v, 4, _NP), d_v, 4, _NU)
    h_p, y2s, y2v = _tc_combine1_dual(
        agg_r1, cr0, cr1, agg_s1, cs0, cs1, x_product, w1_rp_r, w1_pp_r,
        b1_rp.reshape(1, -1), b1_pp.reshape(1, -1), w2_pp_l, w2_pu_l)
    h_u, y2r = _tc_combine1_single(agg_v1, cv0, cv1, x_user, w1_pu_r,
                                   b1_pu.reshape(1, -1), w2_rp_l)

    agg_r2 = _sc_agg(y2r.reshape(-1, 32), _adj(s_r, 2, _NU), d_r, 2, _NP)
    agg_s2 = _sc_agg(y2s.reshape(-1, 32), _adj(s_s, 2, _NP), d_s, 2, _NP)
    agg_v2 = _sc_agg(y2v.reshape(-1, 32), _adj(s_v, 2, _NP), d_v, 2, _NU)
    out_p = _tc_combine2_dual(agg_r2, cr0, cr1, agg_s2, cs0, cs1, h_p,
                              w2_rp_r, w2_pp_r,
                              b2_rp.reshape(1, -1), b2_pp.reshape(1, -1))
    out_u = _tc_combine2_single(agg_v2, cv0, cv1, h_u, w2_pu_r,
                                b2_pu.reshape(1, -1))
    return (out_u, out_p)
